# baseline (speedup 1.0000x reference)
import functools

import jax
import jax.numpy as jnp
import numpy as np
from jax import lax
from jax.experimental import pallas as pl
from jax.experimental.pallas import tpu as pltpu
from jax.experimental.pallas import tpu_sc as plsc

N = 10000
E = 160000
IN = 16
ED = 4
H = 16
OUT = 32
G = 256

NW = 32
CH = 128
E_PAD = 163840
PER_W = E_PAD // NW
NHALF = N // 2
LROWS = 5120
STRIPE = LROWS // 16
NSPLIT = 2 * LROWS
LDUMMY = LROWS
TE = 2048
TNP = 1024


def _rs_mats(i_ch, o_ch):
    c = np.arange(i_ch * o_ch)
    r = (c[None, :] // o_ch == np.arange(i_ch)[:, None]).astype(np.float32)
    s = (c[:, None] % o_ch == np.arange(o_ch)[None, :]).astype(np.float32)
    return jnp.asarray(r), jnp.asarray(s)


def _split_pad(a, fill):
    pad = jnp.full((LROWS - NHALF,) + a.shape[1:], fill, a.dtype)
    return jnp.concatenate([a[:NHALF], pad, a[NHALF:], pad])


def _sc_gather(table, idx2, d):
    mesh = plsc.VectorSubcoreMesh(core_axis_name="c", subcore_axis_name="s")
    nch = PER_W // CH

    @functools.partial(
        pl.kernel, mesh=mesh,
        out_type=jax.ShapeDtypeStruct((E_PAD, d), jnp.float32),
        compiler_params=pltpu.CompilerParams(use_tc_tiling_on_sc=False),
        scratch_types=[
            pltpu.VMEM((nch, CH), jnp.int32),
            pltpu.VMEM((PER_W, d), jnp.float32),
            pltpu.SemaphoreType.DMA,
        ],
    )
    def k(table_hbm, idx_hbm, out_hbm, idx_v, rows_v, sem):
        wid = lax.axis_index("s") * 2 + lax.axis_index("c")
        pltpu.sync_copy(idx_hbm.at[pl.ds(wid * nch, nch)], idx_v)

        def fire(j, carry):
            pltpu.async_copy(table_hbm.at[idx_v.at[j]],
                             rows_v.at[pl.ds(j * CH, CH)], sem)
            return carry

        def drain(j, carry):
            pltpu.make_async_copy(table_hbm.at[idx_v.at[j]],
                                  rows_v.at[pl.ds(j * CH, CH)], sem).wait()
            return carry

        lax.fori_loop(0, nch, fire, 0)
        lax.fori_loop(0, nch, drain, 0)
        pltpu.sync_copy(rows_v, out_hbm.at[pl.ds(wid * PER_W, PER_W)])

    return k(table, idx2)


def _sc_scatter_combine(msg, dst2, r_pad, zeros_hbm, o_ch, relu):
    mesh = plsc.VectorSubcoreMesh(core_axis_name="c", subcore_axis_name="s")
    e_tile = E_PAD // 16
    p_rows = 5120 if o_ch <= 16 else 2560
    p_ch = p_rows // CH
    npass = e_tile // p_rows
    nvec = o_ch // 16

    @functools.partial(
        pl.kernel, mesh=mesh,
        out_type=jax.ShapeDtypeStruct((2, LROWS, o_ch), jnp.float32),
        compiler_params=pltpu.CompilerParams(use_tc_tiling_on_sc=False),
        scratch_types=[
            pltpu.VMEM((p_ch, CH), jnp.int32),
            pltpu.VMEM((p_rows, o_ch), jnp.float32),
            pltpu.VMEM((STRIPE, o_ch), jnp.float32),
            pltpu.VMEM((STRIPE, o_ch), jnp.float32),
            pltpu.VMEM_SHARED((LROWS + 8, o_ch), jnp.float32),
            pltpu.SemaphoreType.DMA,
        ],
    )
    def k(msg_hbm, dst_hbm, r_hbm, z_hbm, out_hbm,
          idx_v, msg_v, acc_v, r_v, acc_sh, sem):
        cid = lax.axis_index("c")
        sid = lax.axis_index("s")
        r0 = sid * STRIPE
        pltpu.sync_copy(z_hbm.at[pl.ds(r0, STRIPE)], acc_sh.at[pl.ds(r0, STRIPE)])
        plsc.subcore_barrier()
        base = cid * NHALF

        for p in range(npass):
            ebase = sid * e_tile + p * p_rows
            pltpu.sync_copy(dst_hbm.at[pl.ds(ebase // CH, p_ch)], idx_v)
            pltpu.sync_copy(msg_hbm.at[pl.ds(ebase, p_rows)], msg_v)

            def remap(j, carry):
                for t in range(CH // 16):
                    v = idx_v[j, pl.ds(t * 16, 16)] - base
                    ok = (v >= 0) & (v < NHALF)
                    idx_v[j, pl.ds(t * 16, 16)] = jnp.where(ok, v, LDUMMY)
                return carry

            def fire(j, carry):
                pltpu.async_copy(msg_v.at[pl.ds(j * CH, CH)],
                                 acc_sh.at[idx_v.at[j]], sem, add=True)
                return carry

            def drain(j, carry):
                pltpu.make_async_copy(msg_v.at[pl.ds(j * CH, CH)],
                                      acc_sh.at[idx_v.at[j]], sem).wait()
                return carry

            lax.fori_loop(0, p_ch, remap, 0)
            lax.fori_loop(0, p_ch, fire, 0)
            lax.fori_loop(0, p_ch, drain, 0)

        plsc.subcore_barrier()
        pltpu.sync_copy(acc_sh.at[pl.ds(r0, STRIPE)], acc_v)
        pltpu.sync_copy(r_hbm.at[pl.ds(cid * LROWS + r0, STRIPE)], r_v)

        def combine(i, carry):
            for t in range(nvec):
                v = acc_v[i, pl.ds(t * 16, 16)] + r_v[i, pl.ds(t * 16, 16)]
                if relu:
                    v = jnp.maximum(v, 0.0)
                acc_v[i, pl.ds(t * 16, 16)] = v
            return carry

        lax.fori_loop(0, STRIPE, combine, 0)
        pltpu.sync_copy(acc_v, out_hbm.at[cid].at[pl.ds(r0, STRIPE)])

    return k(msg, dst2, r_pad, zeros_hbm)


def _dense_msgs(attr, xj, h_prev, w1, b1, w2, b2, root, bias, r_m, s_m, o_ch):
    io = w2.shape[1]
    nr = NSPLIT // TNP

    def body(attr_ref, xj_ref, h_ref, w1_ref, b1_ref, w2_ref, b2_ref,
             root_ref, bias_ref, r_ref, s_ref, out_ref, rout_ref):
        a = attr_ref[...]
        h = jnp.maximum(
            jnp.dot(a, w1_ref[...], preferred_element_type=jnp.float32)
            + b1_ref[...], 0.0)
        w = jnp.dot(h.astype(jnp.bfloat16), w2_ref[...].astype(jnp.bfloat16),
                    preferred_element_type=jnp.float32) + b2_ref[...]
        xr = jnp.dot(xj_ref[...], r_ref[...],
                     preferred_element_type=jnp.float32)
        prod = (w * xr).astype(jnp.bfloat16)
        out_ref[...] = jnp.dot(prod, s_ref[...].astype(jnp.bfloat16),
                               preferred_element_type=jnp.float32)

        @pl.when(pl.program_id(0) < nr)
        def _():
            rout_ref[...] = jnp.dot(
                h_ref[...], root_ref[...],
                preferred_element_type=jnp.float32) + bias_ref[...]

    return pl.pallas_call(
        body,
        grid=(E_PAD // TE,),
        in_specs=[
            pl.BlockSpec((TE, ED), lambda i: (i, 0)),
            pl.BlockSpec((TE, IN), lambda i: (i, 0)),
            pl.BlockSpec((TNP, IN), lambda i: (jnp.minimum(i, nr - 1), 0)),
            pl.BlockSpec((ED, 256), lambda i: (0, 0)),
            pl.BlockSpec((1, 256), lambda i: (0, 0)),
            pl.BlockSpec((256, io), lambda i: (0, 0)),
            pl.BlockSpec((1, io), lambda i: (0, 0)),
            pl.BlockSpec((IN, o_ch), lambda i: (0, 0)),
            pl.BlockSpec((1, o_ch), lambda i: (0, 0)),
            pl.BlockSpec((IN, io), lambda i: (0, 0)),
            pl.BlockSpec((io, o_ch), lambda i: (0, 0)),
        ],
        out_specs=[
            pl.BlockSpec((TE, o_ch), lambda i: (i, 0)),
            pl.BlockSpec((TNP, o_ch), lambda i: (jnp.minimum(i, nr - 1), 0)),
        ],
        out_shape=[
            jax.ShapeDtypeStruct((E_PAD, o_ch), jnp.float32),
            jax.ShapeDtypeStruct((NSPLIT, o_ch), jnp.float32),
        ],
    )(attr, xj, h_prev, w1, b1.reshape(1, -1), w2, b2.reshape(1, -1),
      root, bias.reshape(1, -1), r_m, s_m)


def _pool(h3, batch3):
    ngrid = NSPLIT // TNP

    def body(h_ref, batch_ref, out_ref, sums_scr, cnt_scr):
        pid = pl.program_id(0)
        b = batch_ref[0]
        gid = lax.broadcasted_iota(jnp.int32, (G, TNP), 0)
        onehot = (gid == b).astype(jnp.float32)
        psum = jnp.dot(onehot, h_ref[...], preferred_element_type=jnp.float32)
        pcnt = jnp.sum(onehot, axis=1, keepdims=True)

        @pl.when(pid == 0)
        def _():
            sums_scr[...] = psum
            cnt_scr[...] = pcnt

        @pl.when(pid != 0)
        def _():
            sums_scr[...] = sums_scr[...] + psum
            cnt_scr[...] = cnt_scr[...] + pcnt

        out_ref[...] = sums_scr[...] / jnp.maximum(cnt_scr[...], 1.0)

    return pl.pallas_call(
        body,
        grid=(ngrid,),
        in_specs=[
            pl.BlockSpec((TNP, OUT), lambda i: (i, 0)),
            pl.BlockSpec((1, 1, TNP), lambda i: (i, 0, 0)),
        ],
        out_specs=pl.BlockSpec((G, OUT), lambda i: (0, 0)),
        out_shape=jax.ShapeDtypeStruct((G, OUT), jnp.float32),
        scratch_shapes=[
            pltpu.VMEM((G, OUT), jnp.float32),
            pltpu.VMEM((G, 1), jnp.float32),
        ],
    )(h3, batch3)


def kernel(x, edge_index, edge_attr, batch,
           en1_W1, en1_b1, en1_W2, en1_b2, root1, bias1,
           en2_W1, en2_b1, en2_W2, en2_b2, root2, bias2,
           en3_W1, en3_b1, en3_W2, en3_b2, root3, bias3):
    src = edge_index[0]
    src_sp = jnp.where(src >= NHALF, src + (LROWS - NHALF), src)
    src2 = jnp.pad(src_sp, (0, E_PAD - E)).reshape(E_PAD // CH, CH)
    dst2 = jnp.pad(edge_index[1], (0, E_PAD - E),
                   constant_values=N).reshape(E_PAD // CH, CH)
    attr = jnp.pad(edge_attr, ((0, E_PAD - E), (0, 0)))
    x_sp = _split_pad(x, 0.0)
    batch3 = _split_pad(batch, G + 8).reshape(NSPLIT // TNP, 1, TNP)
    z16 = jnp.zeros((LROWS, H), jnp.float32)
    z32 = jnp.zeros((LROWS, OUT), jnp.float32)
    r1, s1 = _rs_mats(IN, H)
    r3, s3 = _rs_mats(H, OUT)

    xj = _sc_gather(x_sp, src2, IN)
    msg, rt = _dense_msgs(attr, xj, x_sp, en1_W1, en1_b1, en1_W2, en1_b2,
                          root1, bias1, r1, s1, H)
    h1 = _sc_scatter_combine(msg, dst2, rt, z16, H, True)
    h1 = h1.reshape(NSPLIT, H)

    xj = _sc_gather(h1, src2, H)
    msg, rt = _dense_msgs(attr, xj, h1, en2_W1, en2_b1, en2_W2, en2_b2,
                          root2, bias2, r1, s1, H)
    h2 = _sc_scatter_combine(msg, dst2, rt, z16, H, True)
    h2 = h2.reshape(NSPLIT, H)

    xj = _sc_gather(h2, src2, H)
    msg, rt = _dense_msgs(attr, xj, h2, en3_W1, en3_b1, en3_W2, en3_b2,
                          root3, bias3, r3, s3, OUT)
    h3 = _sc_scatter_combine(msg, dst2, rt, z32, OUT, False)
    return _pool(h3.reshape(NSPLIT, OUT), batch3)


# speedup vs baseline: 1.2711x; 1.2711x over previous
import functools

import jax
import jax.numpy as jnp
import numpy as np
from jax import lax
from jax.experimental import pallas as pl
from jax.experimental.pallas import tpu as pltpu
from jax.experimental.pallas import tpu_sc as plsc

N = 10000
E = 160000
IN = 16
ED = 4
H = 16
OUT = 32
G = 256

NW = 32
CH = 128
E_PAD = 163840
PER_W = E_PAD // NW
N_PAD = 10240
TN = 1000
TNP = 1024
TE = 2048


def _rs_mats(i_ch, o_ch):
    c = np.arange(i_ch * o_ch)
    r = (c[None, :] // o_ch == np.arange(i_ch)[:, None]).astype(np.float32)
    s = (c[:, None] % o_ch == np.arange(o_ch)[None, :]).astype(np.float32)
    return jnp.asarray(r), jnp.asarray(s)


def _sc_gather(table, idx2, d):
    mesh = plsc.VectorSubcoreMesh(core_axis_name="c", subcore_axis_name="s")
    nch = PER_W // CH

    @functools.partial(
        pl.kernel, mesh=mesh,
        out_type=jax.ShapeDtypeStruct((E_PAD, d), jnp.float32),
        compiler_params=pltpu.CompilerParams(use_tc_tiling_on_sc=False),
        scratch_types=[
            pltpu.VMEM((nch, CH), jnp.int32),
            pltpu.VMEM((PER_W, d), jnp.float32),
            pltpu.SemaphoreType.DMA,
        ],
    )
    def k(table_hbm, idx_hbm, out_hbm, idx_v, rows_v, sem):
        wid = lax.axis_index("s") * 2 + lax.axis_index("c")
        pltpu.sync_copy(idx_hbm.at[pl.ds(wid * nch, nch)], idx_v)

        def fire(j, carry):
            pltpu.async_copy(table_hbm.at[idx_v.at[j]],
                             rows_v.at[pl.ds(j * CH, CH)], sem)
            return carry

        def drain(j, carry):
            pltpu.make_async_copy(table_hbm.at[idx_v.at[j]],
                                  rows_v.at[pl.ds(j * CH, CH)], sem).wait()
            return carry

        lax.fori_loop(0, nch, fire, 0)
        lax.fori_loop(0, nch, drain, 0)
        pltpu.sync_copy(rows_v, out_hbm.at[pl.ds(wid * PER_W, PER_W)])

    return k(table, idx2)


def _sc_combine_gather(agg, r_pad, idx2):
    mesh = plsc.VectorSubcoreMesh(core_axis_name="c", subcore_axis_name="s")
    nch = PER_W // CH
    stripe = N_PAD // 16

    @functools.partial(
        pl.kernel, mesh=mesh,
        out_type=[jax.ShapeDtypeStruct((N_PAD, H), jnp.float32),
                  jax.ShapeDtypeStruct((E_PAD, H), jnp.float32)],
        compiler_params=pltpu.CompilerParams(use_tc_tiling_on_sc=False),
        scratch_types=[
            pltpu.VMEM((stripe, H), jnp.float32),
            pltpu.VMEM((stripe, H), jnp.float32),
            pltpu.VMEM((stripe, H), jnp.float32),
            pltpu.VMEM((nch, CH), jnp.int32),
            pltpu.VMEM((PER_W, H), jnp.float32),
            pltpu.VMEM_SHARED((N_PAD, H), jnp.float32),
            pltpu.SemaphoreType.DMA,
        ],
    )
    def k(agg_hbm, r_hbm, idx_hbm, h_hbm, xj_hbm,
          a0_v, a1_v, r_v, idx_v, rows_v, h_sh, sem):
        cid = lax.axis_index("c")
        sid = lax.axis_index("s")
        r0 = sid * stripe
        pltpu.sync_copy(agg_hbm.at[0].at[pl.ds(r0, stripe)], a0_v)
        pltpu.sync_copy(agg_hbm.at[1].at[pl.ds(r0, stripe)], a1_v)
        pltpu.sync_copy(r_hbm.at[pl.ds(r0, stripe)], r_v)

        def combine(i, carry):
            a0_v[i] = jnp.maximum(a0_v[i] + a1_v[i] + r_v[i], 0.0)
            return carry

        lax.fori_loop(0, stripe, combine, 0)
        pltpu.sync_copy(a0_v, h_sh.at[pl.ds(r0, stripe)])

        @pl.when(cid == 0)
        def _():
            pltpu.sync_copy(a0_v, h_hbm.at[pl.ds(r0, stripe)])

        plsc.subcore_barrier()

        wid = sid * 2 + cid
        pltpu.sync_copy(idx_hbm.at[pl.ds(wid * nch, nch)], idx_v)

        def fire(j, carry):
            pltpu.async_copy(h_sh.at[idx_v.at[j]],
                             rows_v.at[pl.ds(j * CH, CH)], sem)
            return carry

        def drain(j, carry):
            pltpu.make_async_copy(h_sh.at[idx_v.at[j]],
                                  rows_v.at[pl.ds(j * CH, CH)], sem).wait()
            return carry

        lax.fori_loop(0, nch, fire, 0)
        lax.fori_loop(0, nch, drain, 0)
        pltpu.sync_copy(rows_v, xj_hbm.at[pl.ds(wid * PER_W, PER_W)])

    return k(agg, r_pad, idx2)


def _sc_scatter_add(msg, dst, o_ch, zeros_hbm):
    mesh = plsc.VectorSubcoreMesh(core_axis_name="c", subcore_axis_name="s")
    stripe = N_PAD // 16

    npass = 2 if o_ch > 16 else 1
    p_rows = PER_W // npass
    p_ch = p_rows // CH

    @functools.partial(
        pl.kernel, mesh=mesh,
        out_type=jax.ShapeDtypeStruct((2, N_PAD, o_ch), jnp.float32),
        compiler_params=pltpu.CompilerParams(use_tc_tiling_on_sc=False),
        scratch_types=[
            pltpu.VMEM((p_ch, CH), jnp.int32),
            pltpu.VMEM((p_rows, o_ch), jnp.float32),
            pltpu.VMEM_SHARED((N_PAD, o_ch), jnp.float32),
            pltpu.SemaphoreType.DMA,
        ],
    )
    def k(msg_hbm, dst_hbm, z_hbm, out_hbm, idx_v, msg_v, acc_sh, sem):
        cid = lax.axis_index("c")
        sid = lax.axis_index("s")
        wid = sid * 2 + cid
        r0 = sid * stripe
        pltpu.sync_copy(z_hbm.at[pl.ds(r0, stripe)], acc_sh.at[pl.ds(r0, stripe)])
        plsc.subcore_barrier()

        for p in range(npass):
            rbase = wid * PER_W + p * p_rows
            pltpu.sync_copy(dst_hbm.at[pl.ds(rbase // CH, p_ch)], idx_v)
            pltpu.sync_copy(msg_hbm.at[pl.ds(rbase, p_rows)], msg_v)

            def fire(j, carry):
                pltpu.async_copy(msg_v.at[pl.ds(j * CH, CH)],
                                 acc_sh.at[idx_v.at[j]], sem, add=True)
                return carry

            def drain(j, carry):
                pltpu.make_async_copy(msg_v.at[pl.ds(j * CH, CH)],
                                      acc_sh.at[idx_v.at[j]], sem).wait()
                return carry

            lax.fori_loop(0, p_ch, fire, 0)
            lax.fori_loop(0, p_ch, drain, 0)

        plsc.subcore_barrier()
        pltpu.sync_copy(acc_sh.at[pl.ds(r0, stripe)],
                        out_hbm.at[cid].at[pl.ds(r0, stripe)])

    return k(msg, dst, zeros_hbm)


def _dense_msgs(attr, xj, w1, b1, w2, b2, r_m, s_m, o_ch,
                h_prev=None, root=None, bias=None):
    io = w2.shape[1]
    fold = h_prev is not None
    nr = N_PAD // TNP

    def body(*refs):
        if fold:
            (attr_ref, xj_ref, w1_ref, b1_ref, w2_ref, b2_ref, r_ref, s_ref,
             h_ref, root_ref, bias_ref, out_ref, rout_ref) = refs
        else:
            (attr_ref, xj_ref, w1_ref, b1_ref, w2_ref, b2_ref, r_ref, s_ref,
             out_ref) = refs
        a = attr_ref[...]
        h = jnp.maximum(
            jnp.dot(a, w1_ref[...], preferred_element_type=jnp.float32)
            + b1_ref[...], 0.0)
        w = jnp.dot(h.astype(jnp.bfloat16), w2_ref[...].astype(jnp.bfloat16),
                    preferred_element_type=jnp.float32) + b2_ref[...]
        xr = jnp.dot(xj_ref[...], r_ref[...],
                     preferred_element_type=jnp.float32)
        prod = (w * xr).astype(jnp.bfloat16)
        out_ref[...] = jnp.dot(prod, s_ref[...].astype(jnp.bfloat16),
                               preferred_element_type=jnp.float32)
        if fold:
            @pl.when(pl.program_id(0) < nr)
            def _():
                rout_ref[...] = jnp.dot(
                    h_ref[...], root_ref[...],
                    preferred_element_type=jnp.float32) + bias_ref[...]

    in_specs = [
        pl.BlockSpec((TE, ED), lambda i: (i, 0)),
        pl.BlockSpec((TE, IN), lambda i: (i, 0)),
        pl.BlockSpec((ED, 256), lambda i: (0, 0)),
        pl.BlockSpec((1, 256), lambda i: (0, 0)),
        pl.BlockSpec((256, io), lambda i: (0, 0)),
        pl.BlockSpec((1, io), lambda i: (0, 0)),
        pl.BlockSpec((IN, io), lambda i: (0, 0)),
        pl.BlockSpec((io, o_ch), lambda i: (0, 0)),
    ]
    out_specs = pl.BlockSpec((TE, o_ch), lambda i: (i, 0))
    out_shape = jax.ShapeDtypeStruct((E_PAD, o_ch), jnp.float32)
    args = [attr, xj, w1, b1.reshape(1, -1), w2, b2.reshape(1, -1), r_m, s_m]
    if fold:
        in_specs += [
            pl.BlockSpec((TNP, IN), lambda i: (jnp.minimum(i, nr - 1), 0)),
            pl.BlockSpec((IN, H), lambda i: (0, 0)),
            pl.BlockSpec((1, H), lambda i: (0, 0)),
        ]
        out_specs = [out_specs,
                     pl.BlockSpec((TNP, H),
                                  lambda i: (jnp.minimum(i, nr - 1), 0))]
        out_shape = [out_shape,
                     jax.ShapeDtypeStruct((N_PAD, H), jnp.float32)]
        args += [h_prev, root, bias.reshape(1, -1)]

    return pl.pallas_call(
        body,
        grid=(E_PAD // TE,),
        in_specs=in_specs,
        out_specs=out_specs,
        out_shape=out_shape,
    )(*args)


def _combine_relu(agg, h_in, root, bias, o_ch):
    def body(agg_ref, h_ref, root_ref, bias_ref, out_ref):
        a = agg_ref[0] + agg_ref[1]
        r = jnp.dot(h_ref[...], root_ref[...],
                    preferred_element_type=jnp.float32)
        out_ref[...] = jnp.maximum(a + r + bias_ref[...], 0.0)

    return pl.pallas_call(
        body,
        grid=(N // TN,),
        in_specs=[
            pl.BlockSpec((2, TN, o_ch), lambda i: (0, i, 0)),
            pl.BlockSpec((TN, h_in.shape[1]), lambda i: (i, 0)),
            pl.BlockSpec(root.shape, lambda i: (0, 0)),
            pl.BlockSpec((1, o_ch), lambda i: (0, 0)),
        ],
        out_specs=pl.BlockSpec((TN, o_ch), lambda i: (i, 0)),
        out_shape=jax.ShapeDtypeStruct((N, o_ch), jnp.float32),
    )(agg, h_in, root, bias.reshape(1, -1))


def _combine_pool(agg, h_in, root, bias, batch3):
    ngrid = N // TN

    def body(agg_ref, h_ref, root_ref, bias_ref, batch_ref, out_ref,
             sums_scr, cnt_scr):
        pid = pl.program_id(0)
        a = agg_ref[0] + agg_ref[1]
        r = jnp.dot(h_ref[...], root_ref[...],
                    preferred_element_type=jnp.float32)
        h3 = a + r + bias_ref[...]
        b = batch_ref[0]
        gid = lax.broadcasted_iota(jnp.int32, (G, TN), 0)
        onehot = (gid == b).astype(jnp.float32)
        psum = jnp.dot(onehot, h3, preferred_element_type=jnp.float32)
        pcnt = jnp.sum(onehot, axis=1, keepdims=True)

        @pl.when(pid == 0)
        def _():
            sums_scr[...] = psum
            cnt_scr[...] = pcnt

        @pl.when(pid != 0)
        def _():
            sums_scr[...] = sums_scr[...] + psum
            cnt_scr[...] = cnt_scr[...] + pcnt

        out_ref[...] = sums_scr[...] / jnp.maximum(cnt_scr[...], 1.0)

    return pl.pallas_call(
        body,
        grid=(ngrid,),
        in_specs=[
            pl.BlockSpec((2, TN, OUT), lambda i: (0, i, 0)),
            pl.BlockSpec((TN, H), lambda i: (i, 0)),
            pl.BlockSpec((H, OUT), lambda i: (0, 0)),
            pl.BlockSpec((1, OUT), lambda i: (0, 0)),
            pl.BlockSpec((1, 1, TN), lambda i: (i, 0, 0)),
        ],
        out_specs=pl.BlockSpec((G, OUT), lambda i: (0, 0)),
        out_shape=jax.ShapeDtypeStruct((G, OUT), jnp.float32),
        scratch_shapes=[
            pltpu.VMEM((G, OUT), jnp.float32),
            pltpu.VMEM((G, 1), jnp.float32),
        ],
    )(agg, h_in, root, bias.reshape(1, -1), batch3)


def kernel(x, edge_index, edge_attr, batch,
           en1_W1, en1_b1, en1_W2, en1_b2, root1, bias1,
           en2_W1, en2_b1, en2_W2, en2_b2, root2, bias2,
           en3_W1, en3_b1, en3_W2, en3_b2, root3, bias3):
    src = jnp.pad(edge_index[0], (0, E_PAD - E)).reshape(E_PAD // CH, CH)
    dst = jnp.pad(edge_index[1], (0, E_PAD - E),
                  constant_values=N).reshape(E_PAD // CH, CH)
    attr = jnp.pad(edge_attr, ((0, E_PAD - E), (0, 0)))
    batch3 = batch.reshape(N // TN, 1, TN)
    z16 = jnp.zeros((N_PAD, H), jnp.float32)
    z32 = jnp.zeros((N_PAD, OUT), jnp.float32)
    r1, s1 = _rs_mats(IN, H)
    r3, s3 = _rs_mats(H, OUT)

    x_pad = jnp.pad(x, ((0, N_PAD - N), (0, 0)))

    xj = _sc_gather(x, src, IN)
    msg, rt = _dense_msgs(attr, xj, en1_W1, en1_b1, en1_W2, en1_b2, r1, s1, H,
                          x_pad, root1, bias1)
    agg = _sc_scatter_add(msg, dst, H, z16)
    h1, xj = _sc_combine_gather(agg, rt, src)

    msg, rt = _dense_msgs(attr, xj, en2_W1, en2_b1, en2_W2, en2_b2, r1, s1, H,
                          h1, root2, bias2)
    agg = _sc_scatter_add(msg, dst, H, z16)
    h2, xj = _sc_combine_gather(agg, rt, src)

    msg = _dense_msgs(attr, xj, en3_W1, en3_b1, en3_W2, en3_b2, r3, s3, OUT)
    agg = _sc_scatter_add(msg, dst, OUT, z32)
    return _combine_pool(agg[:, :N], h2[:N], root3, bias3, batch3)


# speedup vs baseline: 1.3569x; 1.0675x over previous
import functools

import jax
import jax.numpy as jnp
import numpy as np
from jax import lax
from jax.experimental import pallas as pl
from jax.experimental.pallas import tpu as pltpu
from jax.experimental.pallas import tpu_sc as plsc

N = 10000
E = 160000
IN = 16
ED = 4
H = 16
OUT = 32
G = 256

NW = 32
CH = 128
E_PAD = 163840
PER_W = E_PAD // NW
N_PAD = 10240
TN = 1000
TNP = 1024
TE = 4096


def _rs_mats(i_ch, o_ch):
    c = np.arange(i_ch * o_ch)
    r = (c[None, :] // o_ch == np.arange(i_ch)[:, None]).astype(np.float32)
    s = (c[:, None] % o_ch == np.arange(o_ch)[None, :]).astype(np.float32)
    return jnp.asarray(r), jnp.asarray(s)


def _sc_gather(table, idx2, d):
    mesh = plsc.VectorSubcoreMesh(core_axis_name="c", subcore_axis_name="s")
    nch = PER_W // CH

    @functools.partial(
        pl.kernel, mesh=mesh,
        out_type=jax.ShapeDtypeStruct((E_PAD, d), jnp.float32),
        compiler_params=pltpu.CompilerParams(use_tc_tiling_on_sc=False),
        scratch_types=[
            pltpu.VMEM((nch, CH), jnp.int32),
            pltpu.VMEM((PER_W, d), jnp.float32),
            pltpu.SemaphoreType.DMA,
        ],
    )
    def k(table_hbm, idx_hbm, out_hbm, idx_v, rows_v, sem):
        wid = lax.axis_index("s") * 2 + lax.axis_index("c")
        pltpu.sync_copy(idx_hbm.at[pl.ds(wid * nch, nch)], idx_v)

        def fire(j, carry):
            pltpu.async_copy(table_hbm.at[idx_v.at[j]],
                             rows_v.at[pl.ds(j * CH, CH)], sem)
            return carry

        def drain(j, carry):
            pltpu.make_async_copy(table_hbm.at[idx_v.at[j]],
                                  rows_v.at[pl.ds(j * CH, CH)], sem).wait()
            return carry

        lax.fori_loop(0, nch, fire, 0)
        lax.fori_loop(0, nch, drain, 0)
        pltpu.sync_copy(rows_v, out_hbm.at[pl.ds(wid * PER_W, PER_W)])

    return k(table, idx2)


def _sc_combine_gather(agg, r_pad, idx2):
    mesh = plsc.VectorSubcoreMesh(core_axis_name="c", subcore_axis_name="s")
    nch = PER_W // CH
    stripe = N_PAD // 16

    @functools.partial(
        pl.kernel, mesh=mesh,
        out_type=[jax.ShapeDtypeStruct((N_PAD, H), jnp.float32),
                  jax.ShapeDtypeStruct((E_PAD, H), jnp.float32)],
        compiler_params=pltpu.CompilerParams(use_tc_tiling_on_sc=False),
        scratch_types=[
            pltpu.VMEM((stripe, H), jnp.float32),
            pltpu.VMEM((stripe, H), jnp.float32),
            pltpu.VMEM((stripe, H), jnp.float32),
            pltpu.VMEM((nch, CH), jnp.int32),
            pltpu.VMEM((PER_W, H), jnp.float32),
            pltpu.VMEM_SHARED((N_PAD, H), jnp.float32),
            pltpu.SemaphoreType.DMA,
        ],
    )
    def k(agg_hbm, r_hbm, idx_hbm, h_hbm, xj_hbm,
          a0_v, a1_v, r_v, idx_v, rows_v, h_sh, sem):
        cid = lax.axis_index("c")
        sid = lax.axis_index("s")
        r0 = sid * stripe
        pltpu.sync_copy(agg_hbm.at[0].at[pl.ds(r0, stripe)], a0_v)
        pltpu.sync_copy(agg_hbm.at[1].at[pl.ds(r0, stripe)], a1_v)
        pltpu.sync_copy(r_hbm.at[pl.ds(r0, stripe)], r_v)

        def combine(i, carry):
            a0_v[i] = jnp.maximum(a0_v[i] + a1_v[i] + r_v[i], 0.0)
            return carry

        lax.fori_loop(0, stripe, combine, 0)
        pltpu.sync_copy(a0_v, h_sh.at[pl.ds(r0, stripe)])

        @pl.when(cid == 0)
        def _():
            pltpu.sync_copy(a0_v, h_hbm.at[pl.ds(r0, stripe)])

        plsc.subcore_barrier()

        wid = sid * 2 + cid
        pltpu.sync_copy(idx_hbm.at[pl.ds(wid * nch, nch)], idx_v)

        def fire(j, carry):
            pltpu.async_copy(h_sh.at[idx_v.at[j]],
                             rows_v.at[pl.ds(j * CH, CH)], sem)
            return carry

        def drain(j, carry):
            pltpu.make_async_copy(h_sh.at[idx_v.at[j]],
                                  rows_v.at[pl.ds(j * CH, CH)], sem).wait()
            return carry

        lax.fori_loop(0, nch, fire, 0)
        lax.fori_loop(0, nch, drain, 0)
        pltpu.sync_copy(rows_v, xj_hbm.at[pl.ds(wid * PER_W, PER_W)])

    return k(agg, r_pad, idx2)


def _sc_scatter_add(msg, dst, o_ch, zeros_hbm):
    mesh = plsc.VectorSubcoreMesh(core_axis_name="c", subcore_axis_name="s")
    stripe = N_PAD // 16

    npass = 2 if o_ch > 16 else 1
    p_rows = PER_W // npass
    p_ch = p_rows // CH

    @functools.partial(
        pl.kernel, mesh=mesh,
        out_type=jax.ShapeDtypeStruct((2, N_PAD, o_ch), jnp.float32),
        compiler_params=pltpu.CompilerParams(use_tc_tiling_on_sc=False),
        scratch_types=[
            pltpu.VMEM((p_ch, CH), jnp.int32),
            pltpu.VMEM((p_rows, o_ch), jnp.float32),
            pltpu.VMEM_SHARED((N_PAD, o_ch), jnp.float32),
            pltpu.SemaphoreType.DMA,
        ],
    )
    def k(msg_hbm, dst_hbm, z_hbm, out_hbm, idx_v, msg_v, acc_sh, sem):
        cid = lax.axis_index("c")
        sid = lax.axis_index("s")
        wid = sid * 2 + cid
        r0 = sid * stripe
        pltpu.sync_copy(z_hbm.at[pl.ds(r0, stripe)], acc_sh.at[pl.ds(r0, stripe)])
        plsc.subcore_barrier()

        for p in range(npass):
            rbase = wid * PER_W + p * p_rows
            pltpu.sync_copy(dst_hbm.at[pl.ds(rbase // CH, p_ch)], idx_v)
            pltpu.sync_copy(msg_hbm.at[pl.ds(rbase, p_rows)], msg_v)

            def fire(j, carry):
                pltpu.async_copy(msg_v.at[pl.ds(j * CH, CH)],
                                 acc_sh.at[idx_v.at[j]], sem, add=True)
                return carry

            def drain(j, carry):
                pltpu.make_async_copy(msg_v.at[pl.ds(j * CH, CH)],
                                      acc_sh.at[idx_v.at[j]], sem).wait()
                return carry

            lax.fori_loop(0, p_ch, fire, 0)
            lax.fori_loop(0, p_ch, drain, 0)

        plsc.subcore_barrier()
        pltpu.sync_copy(acc_sh.at[pl.ds(r0, stripe)],
                        out_hbm.at[cid].at[pl.ds(r0, stripe)])

    return k(msg, dst, zeros_hbm)


def _dense_msgs(attr, xj, w1, b1, w2, b2, r_m, s_m, o_ch,
                h_prev=None, root=None, bias=None):
    io = w2.shape[1]
    fold = h_prev is not None
    nr = N_PAD // TNP

    def body(*refs):
        if fold:
            (attr_ref, xj_ref, w1_ref, b1_ref, w2_ref, b2_ref, r_ref, s_ref,
             h_ref, root_ref, bias_ref, out_ref, rout_ref) = refs
        else:
            (attr_ref, xj_ref, w1_ref, b1_ref, w2_ref, b2_ref, r_ref, s_ref,
             out_ref) = refs
        a = attr_ref[...]
        h = jnp.maximum(
            jnp.dot(a, w1_ref[...], preferred_element_type=jnp.float32)
            + b1_ref[...], 0.0)
        w = jnp.dot(h.astype(jnp.bfloat16), w2_ref[...].astype(jnp.bfloat16),
                    preferred_element_type=jnp.float32) + b2_ref[...]
        xr = jnp.dot(xj_ref[...], r_ref[...],
                     preferred_element_type=jnp.float32)
        prod = (w * xr).astype(jnp.bfloat16)
        out_ref[...] = jnp.dot(prod, s_ref[...].astype(jnp.bfloat16),
                               preferred_element_type=jnp.float32)
        if fold:
            @pl.when(pl.program_id(0) < nr)
            def _():
                rout_ref[...] = jnp.dot(
                    h_ref[...], root_ref[...],
                    preferred_element_type=jnp.float32) + bias_ref[...]

    in_specs = [
        pl.BlockSpec((TE, ED), lambda i: (i, 0)),
        pl.BlockSpec((TE, IN), lambda i: (i, 0)),
        pl.BlockSpec((ED, 256), lambda i: (0, 0)),
        pl.BlockSpec((1, 256), lambda i: (0, 0)),
        pl.BlockSpec((256, io), lambda i: (0, 0)),
        pl.BlockSpec((1, io), lambda i: (0, 0)),
        pl.BlockSpec((IN, io), lambda i: (0, 0)),
        pl.BlockSpec((io, o_ch), lambda i: (0, 0)),
    ]
    out_specs = pl.BlockSpec((TE, o_ch), lambda i: (i, 0))
    out_shape = jax.ShapeDtypeStruct((E_PAD, o_ch), jnp.float32)
    args = [attr, xj, w1, b1.reshape(1, -1), w2, b2.reshape(1, -1), r_m, s_m]
    if fold:
        in_specs += [
            pl.BlockSpec((TNP, IN), lambda i: (jnp.minimum(i, nr - 1), 0)),
            pl.BlockSpec((IN, H), lambda i: (0, 0)),
            pl.BlockSpec((1, H), lambda i: (0, 0)),
        ]
        out_specs = [out_specs,
                     pl.BlockSpec((TNP, H),
                                  lambda i: (jnp.minimum(i, nr - 1), 0))]
        out_shape = [out_shape,
                     jax.ShapeDtypeStruct((N_PAD, H), jnp.float32)]
        args += [h_prev, root, bias.reshape(1, -1)]

    return pl.pallas_call(
        body,
        grid=(E_PAD // TE,),
        in_specs=in_specs,
        out_specs=out_specs,
        out_shape=out_shape,
    )(*args)


def _combine_relu(agg, h_in, root, bias, o_ch):
    def body(agg_ref, h_ref, root_ref, bias_ref, out_ref):
        a = agg_ref[0] + agg_ref[1]
        r = jnp.dot(h_ref[...], root_ref[...],
                    preferred_element_type=jnp.float32)
        out_ref[...] = jnp.maximum(a + r + bias_ref[...], 0.0)

    return pl.pallas_call(
        body,
        grid=(N // TN,),
        in_specs=[
            pl.BlockSpec((2, TN, o_ch), lambda i: (0, i, 0)),
            pl.BlockSpec((TN, h_in.shape[1]), lambda i: (i, 0)),
            pl.BlockSpec(root.shape, lambda i: (0, 0)),
            pl.BlockSpec((1, o_ch), lambda i: (0, 0)),
        ],
        out_specs=pl.BlockSpec((TN, o_ch), lambda i: (i, 0)),
        out_shape=jax.ShapeDtypeStruct((N, o_ch), jnp.float32),
    )(agg, h_in, root, bias.reshape(1, -1))


def _combine_pool(agg, h_in, root, bias, batch3):
    ngrid = N // TN

    def body(agg_ref, h_ref, root_ref, bias_ref, batch_ref, out_ref,
             sums_scr, cnt_scr):
        pid = pl.program_id(0)
        a = agg_ref[0] + agg_ref[1]
        r = jnp.dot(h_ref[...], root_ref[...],
                    preferred_element_type=jnp.float32)
        h3 = a + r + bias_ref[...]
        b = batch_ref[0]
        gid = lax.broadcasted_iota(jnp.int32, (G, TN), 0)
        onehot = (gid == b).astype(jnp.float32)
        psum = jnp.dot(onehot, h3, preferred_element_type=jnp.float32)
        pcnt = jnp.sum(onehot, axis=1, keepdims=True)

        @pl.when(pid == 0)
        def _():
            sums_scr[...] = psum
            cnt_scr[...] = pcnt

        @pl.when(pid != 0)
        def _():
            sums_scr[...] = sums_scr[...] + psum
            cnt_scr[...] = cnt_scr[...] + pcnt

        out_ref[...] = sums_scr[...] / jnp.maximum(cnt_scr[...], 1.0)

    return pl.pallas_call(
        body,
        grid=(ngrid,),
        in_specs=[
            pl.BlockSpec((2, TN, OUT), lambda i: (0, i, 0)),
            pl.BlockSpec((TN, H), lambda i: (i, 0)),
            pl.BlockSpec((H, OUT), lambda i: (0, 0)),
            pl.BlockSpec((1, OUT), lambda i: (0, 0)),
            pl.BlockSpec((1, 1, TN), lambda i: (i, 0, 0)),
        ],
        out_specs=pl.BlockSpec((G, OUT), lambda i: (0, 0)),
        out_shape=jax.ShapeDtypeStruct((G, OUT), jnp.float32),
        scratch_shapes=[
            pltpu.VMEM((G, OUT), jnp.float32),
            pltpu.VMEM((G, 1), jnp.float32),
        ],
    )(agg, h_in, root, bias.reshape(1, -1), batch3)


def kernel(x, edge_index, edge_attr, batch,
           en1_W1, en1_b1, en1_W2, en1_b2, root1, bias1,
           en2_W1, en2_b1, en2_W2, en2_b2, root2, bias2,
           en3_W1, en3_b1, en3_W2, en3_b2, root3, bias3):
    src = jnp.pad(edge_index[0], (0, E_PAD - E)).reshape(E_PAD // CH, CH)
    dst = jnp.pad(edge_index[1], (0, E_PAD - E),
                  constant_values=N).reshape(E_PAD // CH, CH)
    attr = jnp.pad(edge_attr, ((0, E_PAD - E), (0, 0)))
    batch3 = batch.reshape(N // TN, 1, TN)
    z16 = jnp.zeros((N_PAD, H), jnp.float32)
    z32 = jnp.zeros((N_PAD, OUT), jnp.float32)
    r1, s1 = _rs_mats(IN, H)
    r3, s3 = _rs_mats(H, OUT)

    x_pad = jnp.pad(x, ((0, N_PAD - N), (0, 0)))

    xj = _sc_gather(x, src, IN)
    msg, rt = _dense_msgs(attr, xj, en1_W1, en1_b1, en1_W2, en1_b2, r1, s1, H,
                          x_pad, root1, bias1)
    agg = _sc_scatter_add(msg, dst, H, z16)
    h1, xj = _sc_combine_gather(agg, rt, src)

    msg, rt = _dense_msgs(attr, xj, en2_W1, en2_b1, en2_W2, en2_b2, r1, s1, H,
                          h1, root2, bias2)
    agg = _sc_scatter_add(msg, dst, H, z16)
    h2, xj = _sc_combine_gather(agg, rt, src)

    msg = _dense_msgs(attr, xj, en3_W1, en3_b1, en3_W2, en3_b2, r3, s3, OUT)
    agg = _sc_scatter_add(msg, dst, OUT, z32)
    return _combine_pool(agg[:, :N], h2[:N], root3, bias3, batch3)


# speedup vs baseline: 1.3956x; 1.0285x over previous
import functools

import jax
import jax.numpy as jnp
import numpy as np
from jax import lax
from jax.experimental import pallas as pl
from jax.experimental.pallas import tpu as pltpu
from jax.experimental.pallas import tpu_sc as plsc

N = 10000
E = 160000
IN = 16
ED = 4
H = 16
OUT = 32
G = 256

NW = 32
CH = 128
E_PAD = 163840
PER_W = E_PAD // NW
N_PAD = 10240
TN = 1000
TNP = 1024
TE = 8192


def _rs_mats(i_ch, o_ch):
    c = np.arange(i_ch * o_ch)
    r = (c[None, :] // o_ch == np.arange(i_ch)[:, None]).astype(np.float32)
    s = (c[:, None] % o_ch == np.arange(o_ch)[None, :]).astype(np.float32)
    return jnp.asarray(r), jnp.asarray(s)


def _sc_gather(table, idx2, d):
    mesh = plsc.VectorSubcoreMesh(core_axis_name="c", subcore_axis_name="s")
    nch = PER_W // CH

    @functools.partial(
        pl.kernel, mesh=mesh,
        out_type=jax.ShapeDtypeStruct((E_PAD, d), jnp.float32),
        compiler_params=pltpu.CompilerParams(use_tc_tiling_on_sc=False),
        scratch_types=[
            pltpu.VMEM((nch, CH), jnp.int32),
            pltpu.VMEM((PER_W, d), jnp.float32),
            pltpu.SemaphoreType.DMA,
        ],
    )
    def k(table_hbm, idx_hbm, out_hbm, idx_v, rows_v, sem):
        wid = lax.axis_index("s") * 2 + lax.axis_index("c")
        pltpu.sync_copy(idx_hbm.at[pl.ds(wid * nch, nch)], idx_v)

        def fire(j, carry):
            pltpu.async_copy(table_hbm.at[idx_v.at[j]],
                             rows_v.at[pl.ds(j * CH, CH)], sem)
            return carry

        def drain(j, carry):
            pltpu.make_async_copy(table_hbm.at[idx_v.at[j]],
                                  rows_v.at[pl.ds(j * CH, CH)], sem).wait()
            return carry

        lax.fori_loop(0, nch, fire, 0)
        lax.fori_loop(0, nch, drain, 0)
        pltpu.sync_copy(rows_v, out_hbm.at[pl.ds(wid * PER_W, PER_W)])

    return k(table, idx2)


def _sc_combine_gather(agg, r_pad, idx2):
    mesh = plsc.VectorSubcoreMesh(core_axis_name="c", subcore_axis_name="s")
    nch = PER_W // CH
    stripe = N_PAD // 16

    @functools.partial(
        pl.kernel, mesh=mesh,
        out_type=[jax.ShapeDtypeStruct((N_PAD, H), jnp.float32),
                  jax.ShapeDtypeStruct((E_PAD, H), jnp.float32)],
        compiler_params=pltpu.CompilerParams(use_tc_tiling_on_sc=False),
        scratch_types=[
            pltpu.VMEM((stripe, H), jnp.float32),
            pltpu.VMEM((stripe, H), jnp.float32),
            pltpu.VMEM((stripe, H), jnp.float32),
            pltpu.VMEM((nch, CH), jnp.int32),
            pltpu.VMEM((PER_W, H), jnp.float32),
            pltpu.VMEM_SHARED((N_PAD, H), jnp.float32),
            pltpu.SemaphoreType.DMA,
        ],
    )
    def k(agg_hbm, r_hbm, idx_hbm, h_hbm, xj_hbm,
          a0_v, a1_v, r_v, idx_v, rows_v, h_sh, sem):
        cid = lax.axis_index("c")
        sid = lax.axis_index("s")
        r0 = sid * stripe
        pltpu.sync_copy(agg_hbm.at[0].at[pl.ds(r0, stripe)], a0_v)
        pltpu.sync_copy(agg_hbm.at[1].at[pl.ds(r0, stripe)], a1_v)
        pltpu.sync_copy(r_hbm.at[pl.ds(r0, stripe)], r_v)

        def combine(i, carry):
            a0_v[i] = jnp.maximum(a0_v[i] + a1_v[i] + r_v[i], 0.0)
            return carry

        lax.fori_loop(0, stripe, combine, 0)
        pltpu.sync_copy(a0_v, h_sh.at[pl.ds(r0, stripe)])

        @pl.when(cid == 0)
        def _():
            pltpu.sync_copy(a0_v, h_hbm.at[pl.ds(r0, stripe)])

        plsc.subcore_barrier()

        wid = sid * 2 + cid
        pltpu.sync_copy(idx_hbm.at[pl.ds(wid * nch, nch)], idx_v)

        def fire(j, carry):
            pltpu.async_copy(h_sh.at[idx_v.at[j]],
                             rows_v.at[pl.ds(j * CH, CH)], sem)
            return carry

        def drain(j, carry):
            pltpu.make_async_copy(h_sh.at[idx_v.at[j]],
                                  rows_v.at[pl.ds(j * CH, CH)], sem).wait()
            return carry

        lax.fori_loop(0, nch, fire, 0)
        lax.fori_loop(0, nch, drain, 0)
        pltpu.sync_copy(rows_v, xj_hbm.at[pl.ds(wid * PER_W, PER_W)])

    return k(agg, r_pad, idx2)


def _sc_scatter_add(msg, dst, o_ch, zeros_hbm):
    mesh = plsc.VectorSubcoreMesh(core_axis_name="c", subcore_axis_name="s")
    stripe = N_PAD // 16

    npass = 2 if o_ch > 16 else 1
    p_rows = PER_W // npass
    p_ch = p_rows // CH

    @functools.partial(
        pl.kernel, mesh=mesh,
        out_type=jax.ShapeDtypeStruct((2, N_PAD, o_ch), jnp.float32),
        compiler_params=pltpu.CompilerParams(use_tc_tiling_on_sc=False),
        scratch_types=[
            pltpu.VMEM((p_ch, CH), jnp.int32),
            pltpu.VMEM((p_rows, o_ch), jnp.float32),
            pltpu.VMEM_SHARED((N_PAD, o_ch), jnp.float32),
            pltpu.SemaphoreType.DMA,
        ],
    )
    def k(msg_hbm, dst_hbm, z_hbm, out_hbm, idx_v, msg_v, acc_sh, sem):
        cid = lax.axis_index("c")
        sid = lax.axis_index("s")
        wid = sid * 2 + cid
        r0 = sid * stripe
        pltpu.sync_copy(z_hbm.at[pl.ds(r0, stripe)], acc_sh.at[pl.ds(r0, stripe)])
        plsc.subcore_barrier()

        for p in range(npass):
            rbase = wid * PER_W + p * p_rows
            pltpu.sync_copy(dst_hbm.at[pl.ds(rbase // CH, p_ch)], idx_v)
            pltpu.sync_copy(msg_hbm.at[pl.ds(rbase, p_rows)], msg_v)

            def fire(j, carry):
                pltpu.async_copy(msg_v.at[pl.ds(j * CH, CH)],
                                 acc_sh.at[idx_v.at[j]], sem, add=True)
                return carry

            def drain(j, carry):
                pltpu.make_async_copy(msg_v.at[pl.ds(j * CH, CH)],
                                      acc_sh.at[idx_v.at[j]], sem).wait()
                return carry

            lax.fori_loop(0, p_ch, fire, 0)
            lax.fori_loop(0, p_ch, drain, 0)

        plsc.subcore_barrier()
        pltpu.sync_copy(acc_sh.at[pl.ds(r0, stripe)],
                        out_hbm.at[cid].at[pl.ds(r0, stripe)])

    return k(msg, dst, zeros_hbm)


def _dense_msgs(attr, xj, w1, b1, w2, b2, r_m, s_m, o_ch,
                h_prev=None, root=None, bias=None):
    io = w2.shape[1]
    fold = h_prev is not None
    nr = N_PAD // TNP

    def body(*refs):
        if fold:
            (attr_ref, xj_ref, w1_ref, b1_ref, w2_ref, b2_ref, r_ref, s_ref,
             h_ref, root_ref, bias_ref, out_ref, rout_ref) = refs
        else:
            (attr_ref, xj_ref, w1_ref, b1_ref, w2_ref, b2_ref, r_ref, s_ref,
             out_ref) = refs
        a = attr_ref[...]
        h = jnp.maximum(
            jnp.dot(a, w1_ref[...], preferred_element_type=jnp.float32)
            + b1_ref[...], 0.0)
        w = jnp.dot(h.astype(jnp.bfloat16), w2_ref[...].astype(jnp.bfloat16),
                    preferred_element_type=jnp.float32) + b2_ref[...]
        xr = jnp.dot(xj_ref[...], r_ref[...],
                     preferred_element_type=jnp.float32)
        prod = (w * xr).astype(jnp.bfloat16)
        out_ref[...] = jnp.dot(prod, s_ref[...].astype(jnp.bfloat16),
                               preferred_element_type=jnp.float32)
        if fold:
            @pl.when(pl.program_id(0) < nr)
            def _():
                rout_ref[...] = jnp.dot(
                    h_ref[...], root_ref[...],
                    preferred_element_type=jnp.float32) + bias_ref[...]

    in_specs = [
        pl.BlockSpec((TE, ED), lambda i: (i, 0)),
        pl.BlockSpec((TE, IN), lambda i: (i, 0)),
        pl.BlockSpec((ED, 256), lambda i: (0, 0)),
        pl.BlockSpec((1, 256), lambda i: (0, 0)),
        pl.BlockSpec((256, io), lambda i: (0, 0)),
        pl.BlockSpec((1, io), lambda i: (0, 0)),
        pl.BlockSpec((IN, io), lambda i: (0, 0)),
        pl.BlockSpec((io, o_ch), lambda i: (0, 0)),
    ]
    out_specs = pl.BlockSpec((TE, o_ch), lambda i: (i, 0))
    out_shape = jax.ShapeDtypeStruct((E_PAD, o_ch), jnp.float32)
    args = [attr, xj, w1, b1.reshape(1, -1), w2, b2.reshape(1, -1), r_m, s_m]
    if fold:
        in_specs += [
            pl.BlockSpec((TNP, IN), lambda i: (jnp.minimum(i, nr - 1), 0)),
            pl.BlockSpec((IN, H), lambda i: (0, 0)),
            pl.BlockSpec((1, H), lambda i: (0, 0)),
        ]
        out_specs = [out_specs,
                     pl.BlockSpec((TNP, H),
                                  lambda i: (jnp.minimum(i, nr - 1), 0))]
        out_shape = [out_shape,
                     jax.ShapeDtypeStruct((N_PAD, H), jnp.float32)]
        args += [h_prev, root, bias.reshape(1, -1)]

    return pl.pallas_call(
        body,
        grid=(E_PAD // TE,),
        in_specs=in_specs,
        out_specs=out_specs,
        out_shape=out_shape,
    )(*args)


def _combine_relu(agg, h_in, root, bias, o_ch):
    def body(agg_ref, h_ref, root_ref, bias_ref, out_ref):
        a = agg_ref[0] + agg_ref[1]
        r = jnp.dot(h_ref[...], root_ref[...],
                    preferred_element_type=jnp.float32)
        out_ref[...] = jnp.maximum(a + r + bias_ref[...], 0.0)

    return pl.pallas_call(
        body,
        grid=(N // TN,),
        in_specs=[
            pl.BlockSpec((2, TN, o_ch), lambda i: (0, i, 0)),
            pl.BlockSpec((TN, h_in.shape[1]), lambda i: (i, 0)),
            pl.BlockSpec(root.shape, lambda i: (0, 0)),
            pl.BlockSpec((1, o_ch), lambda i: (0, 0)),
        ],
        out_specs=pl.BlockSpec((TN, o_ch), lambda i: (i, 0)),
        out_shape=jax.ShapeDtypeStruct((N, o_ch), jnp.float32),
    )(agg, h_in, root, bias.reshape(1, -1))


def _combine_pool(agg, h_in, root, bias, batch3):
    ngrid = N // TN

    def body(agg_ref, h_ref, root_ref, bias_ref, batch_ref, out_ref,
             sums_scr, cnt_scr):
        pid = pl.program_id(0)
        a = agg_ref[0] + agg_ref[1]
        r = jnp.dot(h_ref[...], root_ref[...],
                    preferred_element_type=jnp.float32)
        h3 = a + r + bias_ref[...]
        b = batch_ref[0]
        gid = lax.broadcasted_iota(jnp.int32, (G, TN), 0)
        onehot = (gid == b).astype(jnp.float32)
        psum = jnp.dot(onehot, h3, preferred_element_type=jnp.float32)
        pcnt = jnp.sum(onehot, axis=1, keepdims=True)

        @pl.when(pid == 0)
        def _():
            sums_scr[...] = psum
            cnt_scr[...] = pcnt

        @pl.when(pid != 0)
        def _():
            sums_scr[...] = sums_scr[...] + psum
            cnt_scr[...] = cnt_scr[...] + pcnt

        out_ref[...] = sums_scr[...] / jnp.maximum(cnt_scr[...], 1.0)

    return pl.pallas_call(
        body,
        grid=(ngrid,),
        in_specs=[
            pl.BlockSpec((2, TN, OUT), lambda i: (0, i, 0)),
            pl.BlockSpec((TN, H), lambda i: (i, 0)),
            pl.BlockSpec((H, OUT), lambda i: (0, 0)),
            pl.BlockSpec((1, OUT), lambda i: (0, 0)),
            pl.BlockSpec((1, 1, TN), lambda i: (i, 0, 0)),
        ],
        out_specs=pl.BlockSpec((G, OUT), lambda i: (0, 0)),
        out_shape=jax.ShapeDtypeStruct((G, OUT), jnp.float32),
        scratch_shapes=[
            pltpu.VMEM((G, OUT), jnp.float32),
            pltpu.VMEM((G, 1), jnp.float32),
        ],
    )(agg, h_in, root, bias.reshape(1, -1), batch3)


def kernel(x, edge_index, edge_attr, batch,
           en1_W1, en1_b1, en1_W2, en1_b2, root1, bias1,
           en2_W1, en2_b1, en2_W2, en2_b2, root2, bias2,
           en3_W1, en3_b1, en3_W2, en3_b2, root3, bias3):
    src = jnp.pad(edge_index[0], (0, E_PAD - E)).reshape(E_PAD // CH, CH)
    dst = jnp.pad(edge_index[1], (0, E_PAD - E),
                  constant_values=N).reshape(E_PAD // CH, CH)
    attr = jnp.pad(edge_attr, ((0, E_PAD - E), (0, 0)))
    batch3 = batch.reshape(N // TN, 1, TN)
    z16 = jnp.zeros((N_PAD, H), jnp.float32)
    z32 = jnp.zeros((N_PAD, OUT), jnp.float32)
    r1, s1 = _rs_mats(IN, H)
    r3, s3 = _rs_mats(H, OUT)

    x_pad = jnp.pad(x, ((0, N_PAD - N), (0, 0)))

    xj = _sc_gather(x, src, IN)
    msg, rt = _dense_msgs(attr, xj, en1_W1, en1_b1, en1_W2, en1_b2, r1, s1, H,
                          x_pad, root1, bias1)
    agg = _sc_scatter_add(msg, dst, H, z16)
    h1, xj = _sc_combine_gather(agg, rt, src)

    msg, rt = _dense_msgs(attr, xj, en2_W1, en2_b1, en2_W2, en2_b2, r1, s1, H,
                          h1, root2, bias2)
    agg = _sc_scatter_add(msg, dst, H, z16)
    h2, xj = _sc_combine_gather(agg, rt, src)

    msg = _dense_msgs(attr, xj, en3_W1, en3_b1, en3_W2, en3_b2, r3, s3, OUT)
    agg = _sc_scatter_add(msg, dst, OUT, z32)
    return _combine_pool(agg[:, :N], h2[:N], root3, bias3, batch3)


# speedup vs baseline: 1.3980x; 1.0017x over previous
import functools

import jax
import jax.numpy as jnp
import numpy as np
from jax import lax
from jax.experimental import pallas as pl
from jax.experimental.pallas import tpu as pltpu
from jax.experimental.pallas import tpu_sc as plsc

N = 10000
E = 160000
IN = 16
ED = 4
H = 16
OUT = 32
G = 256

NW = 32
CH = 128
E_PAD = 163840
PER_W = E_PAD // NW
N_PAD = 10240
TN = 1000
TNP = 1024
TE = 8192


def _rs_mats(i_ch, o_ch):
    c = np.arange(i_ch * o_ch)
    r = (c[None, :] // o_ch == np.arange(i_ch)[:, None]).astype(np.float32)
    s = (c[:, None] % o_ch == np.arange(o_ch)[None, :]).astype(np.float32)
    return jnp.asarray(r), jnp.asarray(s)


def _sc_gather(table, idx2, d):
    mesh = plsc.VectorSubcoreMesh(core_axis_name="c", subcore_axis_name="s")
    nch = PER_W // CH

    @functools.partial(
        pl.kernel, mesh=mesh,
        out_type=jax.ShapeDtypeStruct((E_PAD, d), jnp.float32),
        compiler_params=pltpu.CompilerParams(use_tc_tiling_on_sc=False),
        scratch_types=[
            pltpu.VMEM((nch, CH), jnp.int32),
            pltpu.VMEM((PER_W, d), jnp.float32),
            pltpu.SemaphoreType.DMA,
        ],
    )
    def k(table_hbm, idx_hbm, out_hbm, idx_v, rows_v, sem):
        wid = lax.axis_index("s") * 2 + lax.axis_index("c")
        pltpu.sync_copy(idx_hbm.at[pl.ds(wid * nch, nch)], idx_v)

        def fire(j, carry):
            pltpu.async_copy(table_hbm.at[idx_v.at[j]],
                             rows_v.at[pl.ds(j * CH, CH)], sem)
            return carry

        def drain(j, carry):
            pltpu.make_async_copy(table_hbm.at[idx_v.at[j]],
                                  rows_v.at[pl.ds(j * CH, CH)], sem).wait()
            return carry

        lax.fori_loop(0, nch, fire, 0)
        lax.fori_loop(0, nch, drain, 0)
        pltpu.sync_copy(rows_v, out_hbm.at[pl.ds(wid * PER_W, PER_W)])

    return k(table, idx2)


def _sc_combine_gather(agg, r_pad, idx2):
    mesh = plsc.VectorSubcoreMesh(core_axis_name="c", subcore_axis_name="s")
    nch = PER_W // CH
    stripe = N_PAD // 16

    @functools.partial(
        pl.kernel, mesh=mesh,
        out_type=[jax.ShapeDtypeStruct((N_PAD, H), jnp.float32),
                  jax.ShapeDtypeStruct((E_PAD, H), jnp.float32)],
        compiler_params=pltpu.CompilerParams(use_tc_tiling_on_sc=False),
        scratch_types=[
            pltpu.VMEM((stripe, H), jnp.float32),
            pltpu.VMEM((stripe, H), jnp.float32),
            pltpu.VMEM((stripe, H), jnp.float32),
            pltpu.VMEM((nch, CH), jnp.int32),
            pltpu.VMEM((PER_W, H), jnp.float32),
            pltpu.VMEM_SHARED((N_PAD, H), jnp.float32),
            pltpu.SemaphoreType.DMA,
        ],
    )
    def k(agg_hbm, r_hbm, idx_hbm, h_hbm, xj_hbm,
          a0_v, a1_v, r_v, idx_v, rows_v, h_sh, sem):
        cid = lax.axis_index("c")
        sid = lax.axis_index("s")
        r0 = sid * stripe
        pltpu.sync_copy(agg_hbm.at[0].at[pl.ds(r0, stripe)], a0_v)
        pltpu.sync_copy(agg_hbm.at[1].at[pl.ds(r0, stripe)], a1_v)
        pltpu.sync_copy(r_hbm.at[pl.ds(r0, stripe)], r_v)

        def combine(i, carry):
            a0_v[i] = jnp.maximum(a0_v[i] + a1_v[i] + r_v[i], 0.0)
            return carry

        lax.fori_loop(0, stripe, combine, 0)
        pltpu.sync_copy(a0_v, h_sh.at[pl.ds(r0, stripe)])

        @pl.when(cid == 0)
        def _():
            pltpu.sync_copy(a0_v, h_hbm.at[pl.ds(r0, stripe)])

        plsc.subcore_barrier()

        wid = sid * 2 + cid
        pltpu.sync_copy(idx_hbm.at[pl.ds(wid * nch, nch)], idx_v)

        def fire(j, carry):
            pltpu.async_copy(h_sh.at[idx_v.at[j]],
                             rows_v.at[pl.ds(j * CH, CH)], sem)
            return carry

        def drain(j, carry):
            pltpu.make_async_copy(h_sh.at[idx_v.at[j]],
                                  rows_v.at[pl.ds(j * CH, CH)], sem).wait()
            return carry

        lax.fori_loop(0, nch, fire, 0)
        lax.fori_loop(0, nch, drain, 0)
        pltpu.sync_copy(rows_v, xj_hbm.at[pl.ds(wid * PER_W, PER_W)])

    return k(agg, r_pad, idx2)


def _sc_scatter_add(msg, dst, o_ch, zeros_hbm):
    mesh = plsc.VectorSubcoreMesh(core_axis_name="c", subcore_axis_name="s")
    stripe = N_PAD // 16

    npass = 2 if o_ch > 16 else 1
    p_rows = PER_W // npass
    p_ch = p_rows // CH

    @functools.partial(
        pl.kernel, mesh=mesh,
        out_type=jax.ShapeDtypeStruct((2, N_PAD, o_ch), jnp.float32),
        compiler_params=pltpu.CompilerParams(use_tc_tiling_on_sc=False),
        scratch_types=[
            pltpu.VMEM((p_ch, CH), jnp.int32),
            pltpu.VMEM((p_rows, o_ch), jnp.float32),
            pltpu.VMEM_SHARED((N_PAD, o_ch), jnp.float32),
            pltpu.SemaphoreType.DMA,
        ],
    )
    def k(msg_hbm, dst_hbm, z_hbm, out_hbm, idx_v, msg_v, acc_sh, sem):
        cid = lax.axis_index("c")
        sid = lax.axis_index("s")
        wid = sid * 2 + cid
        r0 = sid * stripe
        pltpu.sync_copy(z_hbm.at[pl.ds(r0, stripe)], acc_sh.at[pl.ds(r0, stripe)])
        plsc.subcore_barrier()

        for p in range(npass):
            rbase = wid * PER_W + p * p_rows
            pltpu.sync_copy(dst_hbm.at[pl.ds(rbase // CH, p_ch)], idx_v)
            pltpu.sync_copy(msg_hbm.at[pl.ds(rbase, p_rows)], msg_v)

            def fire(j, carry):
                pltpu.async_copy(msg_v.at[pl.ds(j * CH, CH)],
                                 acc_sh.at[idx_v.at[j]], sem, add=True)
                return carry

            def drain(j, carry):
                pltpu.make_async_copy(msg_v.at[pl.ds(j * CH, CH)],
                                      acc_sh.at[idx_v.at[j]], sem).wait()
                return carry

            lax.fori_loop(0, p_ch, fire, 0)
            lax.fori_loop(0, p_ch, drain, 0)

        plsc.subcore_barrier()
        pltpu.sync_copy(acc_sh.at[pl.ds(r0, stripe)],
                        out_hbm.at[cid].at[pl.ds(r0, stripe)])

    return k(msg, dst, zeros_hbm)


def _dense_msgs(attr, xj, w1, b1, w2, b2, r_m, s_m, o_ch,
                h_prev=None, root=None, bias=None):
    io = w2.shape[1]
    fold = h_prev is not None
    nr = N_PAD // TNP

    def body(*refs):
        if fold:
            (attr_ref, xj_ref, w1_ref, b1_ref, w2_ref, b2_ref, r_ref, s_ref,
             h_ref, root_ref, bias_ref, out_ref, rout_ref) = refs
        else:
            (attr_ref, xj_ref, w1_ref, b1_ref, w2_ref, b2_ref, r_ref, s_ref,
             out_ref) = refs
        a = attr_ref[...].astype(jnp.bfloat16)
        h = jnp.maximum(
            jnp.dot(a, w1_ref[...].astype(jnp.bfloat16),
                    preferred_element_type=jnp.float32)
            + b1_ref[...], 0.0)
        w = jnp.dot(h.astype(jnp.bfloat16), w2_ref[...].astype(jnp.bfloat16),
                    preferred_element_type=jnp.float32) + b2_ref[...]
        xr = jnp.dot(xj_ref[...].astype(jnp.bfloat16),
                     r_ref[...].astype(jnp.bfloat16),
                     preferred_element_type=jnp.float32)
        prod = w.astype(jnp.bfloat16) * xr.astype(jnp.bfloat16)
        out_ref[...] = jnp.dot(prod, s_ref[...].astype(jnp.bfloat16),
                               preferred_element_type=jnp.float32)
        if fold:
            @pl.when(pl.program_id(0) < nr)
            def _():
                rout_ref[...] = jnp.dot(
                    h_ref[...], root_ref[...],
                    preferred_element_type=jnp.float32) + bias_ref[...]

    in_specs = [
        pl.BlockSpec((TE, ED), lambda i: (i, 0)),
        pl.BlockSpec((TE, IN), lambda i: (i, 0)),
        pl.BlockSpec((ED, 256), lambda i: (0, 0)),
        pl.BlockSpec((1, 256), lambda i: (0, 0)),
        pl.BlockSpec((256, io), lambda i: (0, 0)),
        pl.BlockSpec((1, io), lambda i: (0, 0)),
        pl.BlockSpec((IN, io), lambda i: (0, 0)),
        pl.BlockSpec((io, o_ch), lambda i: (0, 0)),
    ]
    out_specs = pl.BlockSpec((TE, o_ch), lambda i: (i, 0))
    out_shape = jax.ShapeDtypeStruct((E_PAD, o_ch), jnp.float32)
    args = [attr, xj, w1, b1.reshape(1, -1), w2, b2.reshape(1, -1), r_m, s_m]
    if fold:
        in_specs += [
            pl.BlockSpec((TNP, IN), lambda i: (jnp.minimum(i, nr - 1), 0)),
            pl.BlockSpec((IN, H), lambda i: (0, 0)),
            pl.BlockSpec((1, H), lambda i: (0, 0)),
        ]
        out_specs = [out_specs,
                     pl.BlockSpec((TNP, H),
                                  lambda i: (jnp.minimum(i, nr - 1), 0))]
        out_shape = [out_shape,
                     jax.ShapeDtypeStruct((N_PAD, H), jnp.float32)]
        args += [h_prev, root, bias.reshape(1, -1)]

    return pl.pallas_call(
        body,
        grid=(E_PAD // TE,),
        in_specs=in_specs,
        out_specs=out_specs,
        out_shape=out_shape,
    )(*args)


def _combine_relu(agg, h_in, root, bias, o_ch):
    def body(agg_ref, h_ref, root_ref, bias_ref, out_ref):
        a = agg_ref[0] + agg_ref[1]
        r = jnp.dot(h_ref[...], root_ref[...],
                    preferred_element_type=jnp.float32)
        out_ref[...] = jnp.maximum(a + r + bias_ref[...], 0.0)

    return pl.pallas_call(
        body,
        grid=(N // TN,),
        in_specs=[
            pl.BlockSpec((2, TN, o_ch), lambda i: (0, i, 0)),
            pl.BlockSpec((TN, h_in.shape[1]), lambda i: (i, 0)),
            pl.BlockSpec(root.shape, lambda i: (0, 0)),
            pl.BlockSpec((1, o_ch), lambda i: (0, 0)),
        ],
        out_specs=pl.BlockSpec((TN, o_ch), lambda i: (i, 0)),
        out_shape=jax.ShapeDtypeStruct((N, o_ch), jnp.float32),
    )(agg, h_in, root, bias.reshape(1, -1))


def _combine_pool(agg, h_in, root, bias, batch3):
    ngrid = N // TN

    def body(agg_ref, h_ref, root_ref, bias_ref, batch_ref, out_ref,
             sums_scr, cnt_scr):
        pid = pl.program_id(0)
        a = agg_ref[0] + agg_ref[1]
        r = jnp.dot(h_ref[...], root_ref[...],
                    preferred_element_type=jnp.float32)
        h3 = a + r + bias_ref[...]
        b = batch_ref[0]
        gid = lax.broadcasted_iota(jnp.int32, (G, TN), 0)
        onehot = (gid == b).astype(jnp.float32)
        psum = jnp.dot(onehot, h3, preferred_element_type=jnp.float32)
        pcnt = jnp.sum(onehot, axis=1, keepdims=True)

        @pl.when(pid == 0)
        def _():
            sums_scr[...] = psum
            cnt_scr[...] = pcnt

        @pl.when(pid != 0)
        def _():
            sums_scr[...] = sums_scr[...] + psum
            cnt_scr[...] = cnt_scr[...] + pcnt

        out_ref[...] = sums_scr[...] / jnp.maximum(cnt_scr[...], 1.0)

    return pl.pallas_call(
        body,
        grid=(ngrid,),
        in_specs=[
            pl.BlockSpec((2, TN, OUT), lambda i: (0, i, 0)),
            pl.BlockSpec((TN, H), lambda i: (i, 0)),
            pl.BlockSpec((H, OUT), lambda i: (0, 0)),
            pl.BlockSpec((1, OUT), lambda i: (0, 0)),
            pl.BlockSpec((1, 1, TN), lambda i: (i, 0, 0)),
        ],
        out_specs=pl.BlockSpec((G, OUT), lambda i: (0, 0)),
        out_shape=jax.ShapeDtypeStruct((G, OUT), jnp.float32),
        scratch_shapes=[
            pltpu.VMEM((G, OUT), jnp.float32),
            pltpu.VMEM((G, 1), jnp.float32),
        ],
    )(agg, h_in, root, bias.reshape(1, -1), batch3)


def kernel(x, edge_index, edge_attr, batch,
           en1_W1, en1_b1, en1_W2, en1_b2, root1, bias1,
           en2_W1, en2_b1, en2_W2, en2_b2, root2, bias2,
           en3_W1, en3_b1, en3_W2, en3_b2, root3, bias3):
    src = jnp.pad(edge_index[0], (0, E_PAD - E)).reshape(E_PAD // CH, CH)
    dst = jnp.pad(edge_index[1], (0, E_PAD - E),
                  constant_values=N).reshape(E_PAD // CH, CH)
    attr = jnp.pad(edge_attr, ((0, E_PAD - E), (0, 0)))
    batch3 = batch.reshape(N // TN, 1, TN)
    z16 = jnp.zeros((N_PAD, H), jnp.float32)
    z32 = jnp.zeros((N_PAD, OUT), jnp.float32)
    r1, s1 = _rs_mats(IN, H)
    r3, s3 = _rs_mats(H, OUT)

    x_pad = jnp.pad(x, ((0, N_PAD - N), (0, 0)))

    xj = _sc_gather(x, src, IN)
    msg, rt = _dense_msgs(attr, xj, en1_W1, en1_b1, en1_W2, en1_b2, r1, s1, H,
                          x_pad, root1, bias1)
    agg = _sc_scatter_add(msg, dst, H, z16)
    h1, xj = _sc_combine_gather(agg, rt, src)

    msg, rt = _dense_msgs(attr, xj, en2_W1, en2_b1, en2_W2, en2_b2, r1, s1, H,
                          h1, root2, bias2)
    agg = _sc_scatter_add(msg, dst, H, z16)
    h2, xj = _sc_combine_gather(agg, rt, src)

    msg = _dense_msgs(attr, xj, en3_W1, en3_b1, en3_W2, en3_b2, r3, s3, OUT)
    agg = _sc_scatter_add(msg, dst, OUT, z32)
    return _combine_pool(agg[:, :N], h2[:N], root3, bias3, batch3)


# speedup vs baseline: 1.4218x; 1.0171x over previous
import functools

import jax
import jax.numpy as jnp
import numpy as np
from jax import lax
from jax.experimental import pallas as pl
from jax.experimental.pallas import tpu as pltpu
from jax.experimental.pallas import tpu_sc as plsc

N = 10000
E = 160000
IN = 16
ED = 4
H = 16
OUT = 32
G = 256

NW = 32
CH = 128
E_PAD = 163840
PER_W = E_PAD // NW
N_PAD = 10240
TN = 1000
TNP = 1024
TE = 8192


def _rs_mats(i_ch, o_ch):
    c = np.arange(i_ch * o_ch)
    r = (c[None, :] // o_ch == np.arange(i_ch)[:, None]).astype(np.float32)
    s = (c[:, None] % o_ch == np.arange(o_ch)[None, :]).astype(np.float32)
    return jnp.asarray(r), jnp.asarray(s)


def _sc_gather(table, idx2, d):
    mesh = plsc.VectorSubcoreMesh(core_axis_name="c", subcore_axis_name="s")
    nch = PER_W // CH

    @functools.partial(
        pl.kernel, mesh=mesh,
        out_type=jax.ShapeDtypeStruct((E_PAD, d), jnp.float32),
        compiler_params=pltpu.CompilerParams(use_tc_tiling_on_sc=False),
        scratch_types=[
            pltpu.VMEM((nch, CH), jnp.int32),
            pltpu.VMEM((PER_W, d), jnp.float32),
            pltpu.SemaphoreType.DMA,
        ],
    )
    def k(table_hbm, idx_hbm, out_hbm, idx_v, rows_v, sem):
        wid = lax.axis_index("s") * 2 + lax.axis_index("c")
        pltpu.sync_copy(idx_hbm.at[pl.ds(wid * nch, nch)], idx_v)

        def fire(j, carry):
            pltpu.async_copy(table_hbm.at[idx_v.at[j]],
                             rows_v.at[pl.ds(j * CH, CH)], sem)
            return carry

        def drain(j, carry):
            pltpu.make_async_copy(table_hbm.at[idx_v.at[j]],
                                  rows_v.at[pl.ds(j * CH, CH)], sem).wait()
            return carry

        lax.fori_loop(0, nch, fire, 0)
        lax.fori_loop(0, nch, drain, 0)
        pltpu.sync_copy(rows_v, out_hbm.at[pl.ds(wid * PER_W, PER_W)])

    return k(table, idx2)


def _sc_combine_gather(agg, r_pad, idx2):
    mesh = plsc.VectorSubcoreMesh(core_axis_name="c", subcore_axis_name="s")
    nch = PER_W // CH
    stripe = N_PAD // 16

    @functools.partial(
        pl.kernel, mesh=mesh,
        out_type=[jax.ShapeDtypeStruct((N_PAD, H), jnp.float32),
                  jax.ShapeDtypeStruct((E_PAD, H), jnp.float32)],
        compiler_params=pltpu.CompilerParams(use_tc_tiling_on_sc=False),
        scratch_types=[
            pltpu.VMEM((stripe, H), jnp.float32),
            pltpu.VMEM((stripe, H), jnp.float32),
            pltpu.VMEM((stripe, H), jnp.float32),
            pltpu.VMEM((nch, CH), jnp.int32),
            pltpu.VMEM((PER_W, H), jnp.float32),
            pltpu.VMEM_SHARED((N_PAD, H), jnp.float32),
            pltpu.SemaphoreType.DMA,
        ],
    )
    def k(agg_hbm, r_hbm, idx_hbm, h_hbm, xj_hbm,
          a0_v, a1_v, r_v, idx_v, rows_v, h_sh, sem):
        cid = lax.axis_index("c")
        sid = lax.axis_index("s")
        r0 = sid * stripe
        pltpu.sync_copy(agg_hbm.at[0].at[pl.ds(r0, stripe)], a0_v)
        pltpu.sync_copy(agg_hbm.at[1].at[pl.ds(r0, stripe)], a1_v)
        pltpu.sync_copy(r_hbm.at[pl.ds(r0, stripe)], r_v)

        def combine(i, carry):
            a0_v[i] = jnp.maximum(a0_v[i] + a1_v[i] + r_v[i], 0.0)
            return carry

        lax.fori_loop(0, stripe, combine, 0)
        pltpu.sync_copy(a0_v, h_sh.at[pl.ds(r0, stripe)])

        @pl.when(cid == 0)
        def _():
            pltpu.sync_copy(a0_v, h_hbm.at[pl.ds(r0, stripe)])

        plsc.subcore_barrier()

        wid = sid * 2 + cid
        pltpu.sync_copy(idx_hbm.at[pl.ds(wid * nch, nch)], idx_v)

        def fire(j, carry):
            pltpu.async_copy(h_sh.at[idx_v.at[j]],
                             rows_v.at[pl.ds(j * CH, CH)], sem)
            return carry

        def drain(j, carry):
            pltpu.make_async_copy(h_sh.at[idx_v.at[j]],
                                  rows_v.at[pl.ds(j * CH, CH)], sem).wait()
            return carry

        lax.fori_loop(0, nch, fire, 0)
        lax.fori_loop(0, nch, drain, 0)
        pltpu.sync_copy(rows_v, xj_hbm.at[pl.ds(wid * PER_W, PER_W)])

    return k(agg, r_pad, idx2)


def _sc_scatter_add(msg, dst, o_ch, zeros_hbm):
    mesh = plsc.VectorSubcoreMesh(core_axis_name="c", subcore_axis_name="s")
    stripe = N_PAD // 16

    npass = 2 if o_ch > 16 else 1
    p_rows = PER_W // npass
    p_ch = p_rows // CH

    @functools.partial(
        pl.kernel, mesh=mesh,
        out_type=jax.ShapeDtypeStruct((2, N_PAD, o_ch), jnp.float32),
        compiler_params=pltpu.CompilerParams(use_tc_tiling_on_sc=False),
        scratch_types=[
            pltpu.VMEM((p_ch, CH), jnp.int32),
            pltpu.VMEM((p_rows, o_ch), jnp.float32),
            pltpu.VMEM_SHARED((N_PAD, o_ch), jnp.float32),
            pltpu.SemaphoreType.DMA,
        ],
    )
    def k(msg_hbm, dst_hbm, z_hbm, out_hbm, idx_v, msg_v, acc_sh, sem):
        cid = lax.axis_index("c")
        sid = lax.axis_index("s")
        wid = sid * 2 + cid
        r0 = sid * stripe
        pltpu.sync_copy(z_hbm.at[pl.ds(r0, stripe)], acc_sh.at[pl.ds(r0, stripe)])
        plsc.subcore_barrier()

        for p in range(npass):
            rbase = wid * PER_W + p * p_rows
            pltpu.sync_copy(dst_hbm.at[pl.ds(rbase // CH, p_ch)], idx_v)
            pltpu.sync_copy(msg_hbm.at[pl.ds(rbase, p_rows)], msg_v)

            def fire(j, carry):
                pltpu.async_copy(msg_v.at[pl.ds(j * CH, CH)],
                                 acc_sh.at[idx_v.at[j]], sem, add=True)
                return carry

            def drain(j, carry):
                pltpu.make_async_copy(msg_v.at[pl.ds(j * CH, CH)],
                                      acc_sh.at[idx_v.at[j]], sem).wait()
                return carry

            lax.fori_loop(0, p_ch, fire, 0)
            lax.fori_loop(0, p_ch, drain, 0)

        plsc.subcore_barrier()
        pltpu.sync_copy(acc_sh.at[pl.ds(r0, stripe)],
                        out_hbm.at[cid].at[pl.ds(r0, stripe)])

    return k(msg, dst, zeros_hbm)


def _dense_msgs(attr_e, xj, w1, b1, w2, b2, r_m, s_m, o_ch,
                h_prev=None, root=None, bias=None):
    io = w2.shape[1]
    fold = h_prev is not None
    nr = N_PAD // TNP
    bf = jnp.bfloat16
    w1e = jnp.concatenate([w1, b1.reshape(1, -1)], axis=0).astype(bf)
    b_m = ((r_m * b2.reshape(1, -1)) @ s_m).astype(bf)

    def body(*refs):
        if fold:
            (attr_ref, xj_ref, w1_ref, w2_ref, r_ref, s_ref, b_ref,
             h_ref, root_ref, bias_ref, out_ref, rout_ref) = refs
        else:
            (attr_ref, xj_ref, w1_ref, w2_ref, r_ref, s_ref, b_ref,
             out_ref) = refs
        h = jnp.maximum(
            jnp.dot(attr_ref[...], w1_ref[...],
                    preferred_element_type=jnp.float32), 0.0)
        w = jnp.dot(h.astype(bf), w2_ref[...],
                    preferred_element_type=jnp.float32)
        xjb = xj_ref[...].astype(bf)
        xr = jnp.dot(xjb, r_ref[...], preferred_element_type=jnp.float32)
        prod = (w * xr).astype(bf)
        out_ref[...] = (
            jnp.dot(prod, s_ref[...], preferred_element_type=jnp.float32)
            + jnp.dot(xjb, b_ref[...], preferred_element_type=jnp.float32))
        if fold:
            @pl.when(pl.program_id(0) < nr)
            def _():
                rout_ref[...] = jnp.dot(
                    h_ref[...], root_ref[...],
                    preferred_element_type=jnp.float32) + bias_ref[...]

    in_specs = [
        pl.BlockSpec((TE, ED + 1), lambda i: (i, 0)),
        pl.BlockSpec((TE, IN), lambda i: (i, 0)),
        pl.BlockSpec((ED + 1, 256), lambda i: (0, 0)),
        pl.BlockSpec((256, io), lambda i: (0, 0)),
        pl.BlockSpec((IN, io), lambda i: (0, 0)),
        pl.BlockSpec((io, o_ch), lambda i: (0, 0)),
        pl.BlockSpec((IN, o_ch), lambda i: (0, 0)),
    ]
    out_specs = pl.BlockSpec((TE, o_ch), lambda i: (i, 0))
    out_shape = jax.ShapeDtypeStruct((E_PAD, o_ch), jnp.float32)
    args = [attr_e, xj, w1e, w2.astype(bf), r_m.astype(bf), s_m.astype(bf),
            b_m]
    if fold:
        in_specs += [
            pl.BlockSpec((TNP, IN), lambda i: (jnp.minimum(i, nr - 1), 0)),
            pl.BlockSpec((IN, H), lambda i: (0, 0)),
            pl.BlockSpec((1, H), lambda i: (0, 0)),
        ]
        out_specs = [out_specs,
                     pl.BlockSpec((TNP, H),
                                  lambda i: (jnp.minimum(i, nr - 1), 0))]
        out_shape = [out_shape,
                     jax.ShapeDtypeStruct((N_PAD, H), jnp.float32)]
        args += [h_prev, root, bias.reshape(1, -1)]

    return pl.pallas_call(
        body,
        grid=(E_PAD // TE,),
        in_specs=in_specs,
        out_specs=out_specs,
        out_shape=out_shape,
    )(*args)


def _combine_relu(agg, h_in, root, bias, o_ch):
    def body(agg_ref, h_ref, root_ref, bias_ref, out_ref):
        a = agg_ref[0] + agg_ref[1]
        r = jnp.dot(h_ref[...], root_ref[...],
                    preferred_element_type=jnp.float32)
        out_ref[...] = jnp.maximum(a + r + bias_ref[...], 0.0)

    return pl.pallas_call(
        body,
        grid=(N // TN,),
        in_specs=[
            pl.BlockSpec((2, TN, o_ch), lambda i: (0, i, 0)),
            pl.BlockSpec((TN, h_in.shape[1]), lambda i: (i, 0)),
            pl.BlockSpec(root.shape, lambda i: (0, 0)),
            pl.BlockSpec((1, o_ch), lambda i: (0, 0)),
        ],
        out_specs=pl.BlockSpec((TN, o_ch), lambda i: (i, 0)),
        out_shape=jax.ShapeDtypeStruct((N, o_ch), jnp.float32),
    )(agg, h_in, root, bias.reshape(1, -1))


def _combine_pool(agg, h_in, root, bias, batch3):
    ngrid = N // TN

    def body(agg_ref, h_ref, root_ref, bias_ref, batch_ref, out_ref,
             sums_scr, cnt_scr):
        pid = pl.program_id(0)
        a = agg_ref[0] + agg_ref[1]
        r = jnp.dot(h_ref[...], root_ref[...],
                    preferred_element_type=jnp.float32)
        h3 = a + r + bias_ref[...]
        b = batch_ref[0]
        gid = lax.broadcasted_iota(jnp.int32, (G, TN), 0)
        onehot = (gid == b).astype(jnp.float32)
        psum = jnp.dot(onehot, h3, preferred_element_type=jnp.float32)
        pcnt = jnp.sum(onehot, axis=1, keepdims=True)

        @pl.when(pid == 0)
        def _():
            sums_scr[...] = psum
            cnt_scr[...] = pcnt

        @pl.when(pid != 0)
        def _():
            sums_scr[...] = sums_scr[...] + psum
            cnt_scr[...] = cnt_scr[...] + pcnt

        out_ref[...] = sums_scr[...] / jnp.maximum(cnt_scr[...], 1.0)

    return pl.pallas_call(
        body,
        grid=(ngrid,),
        in_specs=[
            pl.BlockSpec((2, TN, OUT), lambda i: (0, i, 0)),
            pl.BlockSpec((TN, H), lambda i: (i, 0)),
            pl.BlockSpec((H, OUT), lambda i: (0, 0)),
            pl.BlockSpec((1, OUT), lambda i: (0, 0)),
            pl.BlockSpec((1, 1, TN), lambda i: (i, 0, 0)),
        ],
        out_specs=pl.BlockSpec((G, OUT), lambda i: (0, 0)),
        out_shape=jax.ShapeDtypeStruct((G, OUT), jnp.float32),
        scratch_shapes=[
            pltpu.VMEM((G, OUT), jnp.float32),
            pltpu.VMEM((G, 1), jnp.float32),
        ],
    )(agg, h_in, root, bias.reshape(1, -1), batch3)


def kernel(x, edge_index, edge_attr, batch,
           en1_W1, en1_b1, en1_W2, en1_b2, root1, bias1,
           en2_W1, en2_b1, en2_W2, en2_b2, root2, bias2,
           en3_W1, en3_b1, en3_W2, en3_b2, root3, bias3):
    src = jnp.pad(edge_index[0], (0, E_PAD - E)).reshape(E_PAD // CH, CH)
    dst = jnp.pad(edge_index[1], (0, E_PAD - E),
                  constant_values=N).reshape(E_PAD // CH, CH)
    attr = jnp.concatenate(
        [jnp.pad(edge_attr, ((0, E_PAD - E), (0, 0))),
         jnp.ones((E_PAD, 1), jnp.float32)], axis=1).astype(jnp.bfloat16)
    batch3 = batch.reshape(N // TN, 1, TN)
    z16 = jnp.zeros((N_PAD, H), jnp.float32)
    z32 = jnp.zeros((N_PAD, OUT), jnp.float32)
    r1, s1 = _rs_mats(IN, H)
    r3, s3 = _rs_mats(H, OUT)

    x_pad = jnp.pad(x, ((0, N_PAD - N), (0, 0)))

    xj = _sc_gather(x, src, IN)
    msg, rt = _dense_msgs(attr, xj, en1_W1, en1_b1, en1_W2, en1_b2, r1, s1, H,
                          x_pad, root1, bias1)
    agg = _sc_scatter_add(msg, dst, H, z16)
    h1, xj = _sc_combine_gather(agg, rt, src)

    msg, rt = _dense_msgs(attr, xj, en2_W1, en2_b1, en2_W2, en2_b2, r1, s1, H,
                          h1, root2, bias2)
    agg = _sc_scatter_add(msg, dst, H, z16)
    h2, xj = _sc_combine_gather(agg, rt, src)

    msg = _dense_msgs(attr, xj, en3_W1, en3_b1, en3_W2, en3_b2, r3, s3, OUT)
    agg = _sc_scatter_add(msg, dst, OUT, z32)
    return _combine_pool(agg[:, :N], h2[:N], root3, bias3, batch3)


# speedup vs baseline: 1.4238x; 1.0014x over previous
import functools

import jax
import jax.numpy as jnp
import numpy as np
from jax import lax
from jax.experimental import pallas as pl
from jax.experimental.pallas import tpu as pltpu
from jax.experimental.pallas import tpu_sc as plsc

N = 10000
E = 160000
IN = 16
ED = 4
H = 16
OUT = 32
G = 256

NW = 32
CH = 128
E_PAD = 163840
PER_W = E_PAD // NW
N_PAD = 10240
TN = 1000
TNP = 1024
TE = 8192


def _rs_mats(i_ch, o_ch):
    c = np.arange(i_ch * o_ch)
    r = (c[None, :] // o_ch == np.arange(i_ch)[:, None]).astype(np.float32)
    s = (c[:, None] % o_ch == np.arange(o_ch)[None, :]).astype(np.float32)
    return jnp.asarray(r), jnp.asarray(s)


def _sc_gather(table, idx2, d):
    mesh = plsc.VectorSubcoreMesh(core_axis_name="c", subcore_axis_name="s")
    nch = PER_W // CH

    @functools.partial(
        pl.kernel, mesh=mesh,
        out_type=jax.ShapeDtypeStruct((E_PAD, d), jnp.float32),
        compiler_params=pltpu.CompilerParams(use_tc_tiling_on_sc=False),
        scratch_types=[
            pltpu.VMEM((nch, CH), jnp.int32),
            pltpu.VMEM((PER_W, d), jnp.float32),
            pltpu.SemaphoreType.DMA,
        ],
    )
    def k(table_hbm, idx_hbm, out_hbm, idx_v, rows_v, sem):
        wid = lax.axis_index("s") * 2 + lax.axis_index("c")
        pltpu.sync_copy(idx_hbm.at[pl.ds(wid * nch, nch)], idx_v)

        def fire(j, carry):
            pltpu.async_copy(table_hbm.at[idx_v.at[j]],
                             rows_v.at[pl.ds(j * CH, CH)], sem)
            return carry

        def drain(j, carry):
            pltpu.make_async_copy(table_hbm.at[idx_v.at[j]],
                                  rows_v.at[pl.ds(j * CH, CH)], sem).wait()
            return carry

        lax.fori_loop(0, nch, fire, 0)
        lax.fori_loop(0, nch, drain, 0)
        pltpu.sync_copy(rows_v, out_hbm.at[pl.ds(wid * PER_W, PER_W)])

    return k(table, idx2)


def _sc_combine_gather(agg, r_pad, idx2):
    mesh = plsc.VectorSubcoreMesh(core_axis_name="c", subcore_axis_name="s")
    nch = PER_W // CH
    stripe = N_PAD // 16

    @functools.partial(
        pl.kernel, mesh=mesh,
        out_type=[jax.ShapeDtypeStruct((N_PAD, H), jnp.float32),
                  jax.ShapeDtypeStruct((E_PAD, H), jnp.float32)],
        compiler_params=pltpu.CompilerParams(use_tc_tiling_on_sc=False),
        scratch_types=[
            pltpu.VMEM((stripe, H), jnp.float32),
            pltpu.VMEM((stripe, H), jnp.float32),
            pltpu.VMEM((stripe, H), jnp.float32),
            pltpu.VMEM((nch, CH), jnp.int32),
            pltpu.VMEM((PER_W, H), jnp.float32),
            pltpu.VMEM_SHARED((N_PAD, H), jnp.float32),
            pltpu.SemaphoreType.DMA,
        ],
    )
    def k(agg_hbm, r_hbm, idx_hbm, h_hbm, xj_hbm,
          a0_v, a1_v, r_v, idx_v, rows_v, h_sh, sem):
        cid = lax.axis_index("c")
        sid = lax.axis_index("s")
        r0 = sid * stripe
        pltpu.sync_copy(agg_hbm.at[0].at[pl.ds(r0, stripe)], a0_v)
        pltpu.sync_copy(agg_hbm.at[1].at[pl.ds(r0, stripe)], a1_v)
        pltpu.sync_copy(r_hbm.at[pl.ds(r0, stripe)], r_v)

        def combine(i, carry):
            a0_v[i] = jnp.maximum(a0_v[i] + a1_v[i] + r_v[i], 0.0)
            return carry

        lax.fori_loop(0, stripe, combine, 0)
        pltpu.sync_copy(a0_v, h_sh.at[pl.ds(r0, stripe)])

        @pl.when(cid == 0)
        def _():
            pltpu.sync_copy(a0_v, h_hbm.at[pl.ds(r0, stripe)])

        plsc.subcore_barrier()

        wid = sid * 2 + cid
        pltpu.sync_copy(idx_hbm.at[pl.ds(wid * nch, nch)], idx_v)

        def fire(j, carry):
            pltpu.async_copy(h_sh.at[idx_v.at[j]],
                             rows_v.at[pl.ds(j * CH, CH)], sem)
            return carry

        def drain(j, carry):
            pltpu.make_async_copy(h_sh.at[idx_v.at[j]],
                                  rows_v.at[pl.ds(j * CH, CH)], sem).wait()
            return carry

        lax.fori_loop(0, nch, fire, 0)
        lax.fori_loop(0, nch, drain, 0)
        pltpu.sync_copy(rows_v, xj_hbm.at[pl.ds(wid * PER_W, PER_W)])

    return k(agg, r_pad, idx2)


def _sc_scatter_add(msg, dst, o_ch, zeros_hbm):
    mesh = plsc.VectorSubcoreMesh(core_axis_name="c", subcore_axis_name="s")
    stripe = N_PAD // 16

    npass = 2 if o_ch > 16 else 1
    p_rows = PER_W // npass
    p_ch = p_rows // CH

    @functools.partial(
        pl.kernel, mesh=mesh,
        out_type=jax.ShapeDtypeStruct((2, N_PAD, o_ch), jnp.float32),
        compiler_params=pltpu.CompilerParams(use_tc_tiling_on_sc=False),
        scratch_types=[
            pltpu.VMEM((p_ch, CH), jnp.int32),
            pltpu.VMEM((p_rows, o_ch), jnp.float32),
            pltpu.VMEM_SHARED((N_PAD, o_ch), jnp.float32),
            pltpu.SemaphoreType.DMA,
        ],
    )
    def k(msg_hbm, dst_hbm, z_hbm, out_hbm, idx_v, msg_v, acc_sh, sem):
        cid = lax.axis_index("c")
        sid = lax.axis_index("s")
        wid = sid * 2 + cid
        r0 = sid * stripe
        pltpu.sync_copy(z_hbm.at[pl.ds(r0, stripe)], acc_sh.at[pl.ds(r0, stripe)])
        plsc.subcore_barrier()

        for p in range(npass):
            rbase = wid * PER_W + p * p_rows
            pltpu.sync_copy(dst_hbm.at[pl.ds(rbase // CH, p_ch)], idx_v)
            pltpu.sync_copy(msg_hbm.at[pl.ds(rbase, p_rows)], msg_v)

            def fire(j, carry):
                pltpu.async_copy(msg_v.at[pl.ds(j * CH, CH)],
                                 acc_sh.at[idx_v.at[j]], sem, add=True)
                return carry

            def drain(j, carry):
                pltpu.make_async_copy(msg_v.at[pl.ds(j * CH, CH)],
                                      acc_sh.at[idx_v.at[j]], sem).wait()
                return carry

            lax.fori_loop(0, p_ch, fire, 0)
            lax.fori_loop(0, p_ch, drain, 0)

        plsc.subcore_barrier()
        pltpu.sync_copy(acc_sh.at[pl.ds(r0, stripe)],
                        out_hbm.at[cid].at[pl.ds(r0, stripe)])

    return k(msg, dst, zeros_hbm)


def _dense_msgs(attr_e, xj, w1, b1, w2, b2, r_m, s_m, o_ch,
                h_prev=None, root=None, bias=None):
    io = w2.shape[1]
    fold = h_prev is not None
    nr = N_PAD // TNP
    bf = jnp.bfloat16
    w1e = jnp.concatenate([w1, b1.reshape(1, -1)], axis=0).astype(bf)
    b_m = ((r_m * b2.reshape(1, -1)) @ s_m).astype(bf)

    def body(*refs):
        if fold:
            (attr_ref, xj_ref, w1_ref, w2_ref, r_ref, s_ref, b_ref,
             h_ref, root_ref, bias_ref, out_ref, rout_ref) = refs
        else:
            (attr_ref, xj_ref, w1_ref, w2_ref, r_ref, s_ref, b_ref,
             out_ref) = refs
        h = jnp.maximum(
            jnp.dot(attr_ref[...], w1_ref[...],
                    preferred_element_type=jnp.float32), 0.0)
        w = jnp.dot(h.astype(bf), w2_ref[...],
                    preferred_element_type=jnp.float32)
        xjb = xj_ref[...].astype(bf)
        xr = jnp.dot(xjb, r_ref[...], preferred_element_type=jnp.float32)
        prod = (w * xr).astype(bf)
        out_ref[...] = (
            jnp.dot(prod, s_ref[...], preferred_element_type=jnp.float32)
            + jnp.dot(xjb, b_ref[...], preferred_element_type=jnp.float32))
        if fold:
            @pl.when(pl.program_id(0) < nr)
            def _():
                rout_ref[...] = jnp.dot(
                    h_ref[...], root_ref[...],
                    preferred_element_type=jnp.float32) + bias_ref[...]

    in_specs = [
        pl.BlockSpec((TE, ED + 1), lambda i: (i, 0)),
        pl.BlockSpec((TE, IN), lambda i: (i, 0)),
        pl.BlockSpec((ED + 1, 256), lambda i: (0, 0)),
        pl.BlockSpec((256, io), lambda i: (0, 0)),
        pl.BlockSpec((IN, io), lambda i: (0, 0)),
        pl.BlockSpec((io, o_ch), lambda i: (0, 0)),
        pl.BlockSpec((IN, o_ch), lambda i: (0, 0)),
    ]
    out_specs = pl.BlockSpec((TE, o_ch), lambda i: (i, 0))
    out_shape = jax.ShapeDtypeStruct((E_PAD, o_ch), jnp.float32)
    args = [attr_e, xj, w1e, w2.astype(bf), r_m.astype(bf), s_m.astype(bf),
            b_m]
    if fold:
        in_specs += [
            pl.BlockSpec((TNP, IN), lambda i: (jnp.minimum(i, nr - 1), 0)),
            pl.BlockSpec((IN, H), lambda i: (0, 0)),
            pl.BlockSpec((1, H), lambda i: (0, 0)),
        ]
        out_specs = [out_specs,
                     pl.BlockSpec((TNP, H),
                                  lambda i: (jnp.minimum(i, nr - 1), 0))]
        out_shape = [out_shape,
                     jax.ShapeDtypeStruct((N_PAD, H), jnp.float32)]
        args += [h_prev, root, bias.reshape(1, -1)]

    return pl.pallas_call(
        body,
        grid=(E_PAD // TE,),
        in_specs=in_specs,
        out_specs=out_specs,
        out_shape=out_shape,
    )(*args)


def _combine_pool(agg, h_in, root, bias, batch3):
    ngrid = N // TN

    def body(agg_ref, h_ref, root_ref, bias_ref, batch_ref, out_ref,
             sums_scr, cnt_scr):
        pid = pl.program_id(0)
        a = agg_ref[0] + agg_ref[1]
        r = jnp.dot(h_ref[...], root_ref[...],
                    preferred_element_type=jnp.float32)
        h3 = a + r + bias_ref[...]
        b = batch_ref[0]
        gid = lax.broadcasted_iota(jnp.int32, (G, TN), 0)
        onehot = (gid == b).astype(jnp.float32)
        psum = jnp.dot(onehot, h3, preferred_element_type=jnp.float32)
        pcnt = jnp.sum(onehot, axis=1, keepdims=True)

        @pl.when(pid == 0)
        def _():
            sums_scr[...] = psum
            cnt_scr[...] = pcnt

        @pl.when(pid != 0)
        def _():
            sums_scr[...] = sums_scr[...] + psum
            cnt_scr[...] = cnt_scr[...] + pcnt

        out_ref[...] = sums_scr[...] / jnp.maximum(cnt_scr[...], 1.0)

    return pl.pallas_call(
        body,
        grid=(ngrid,),
        in_specs=[
            pl.BlockSpec((2, TN, OUT), lambda i: (0, i, 0)),
            pl.BlockSpec((TN, H), lambda i: (i, 0)),
            pl.BlockSpec((H, OUT), lambda i: (0, 0)),
            pl.BlockSpec((1, OUT), lambda i: (0, 0)),
            pl.BlockSpec((1, 1, TN), lambda i: (i, 0, 0)),
        ],
        out_specs=pl.BlockSpec((G, OUT), lambda i: (0, 0)),
        out_shape=jax.ShapeDtypeStruct((G, OUT), jnp.float32),
        scratch_shapes=[
            pltpu.VMEM((G, OUT), jnp.float32),
            pltpu.VMEM((G, 1), jnp.float32),
        ],
    )(agg, h_in, root, bias.reshape(1, -1), batch3)


def kernel(x, edge_index, edge_attr, batch,
           en1_W1, en1_b1, en1_W2, en1_b2, root1, bias1,
           en2_W1, en2_b1, en2_W2, en2_b2, root2, bias2,
           en3_W1, en3_b1, en3_W2, en3_b2, root3, bias3):
    src = jnp.pad(edge_index[0], (0, E_PAD - E)).reshape(E_PAD // CH, CH)
    dst = jnp.pad(edge_index[1], (0, E_PAD - E),
                  constant_values=N).reshape(E_PAD // CH, CH)
    attr = jnp.concatenate(
        [jnp.pad(edge_attr, ((0, E_PAD - E), (0, 0))),
         jnp.ones((E_PAD, 1), jnp.float32)], axis=1).astype(jnp.bfloat16)
    batch3 = batch.reshape(N // TN, 1, TN)
    z16 = jnp.zeros((N_PAD, H), jnp.float32)
    z32 = jnp.zeros((N_PAD, OUT), jnp.float32)
    r1, s1 = _rs_mats(IN, H)
    r3, s3 = _rs_mats(H, OUT)

    x_pad = jnp.pad(x, ((0, N_PAD - N), (0, 0)))

    xj = _sc_gather(x, src, IN)
    msg, rt = _dense_msgs(attr, xj, en1_W1, en1_b1, en1_W2, en1_b2, r1, s1, H,
                          x_pad, root1, bias1)
    agg = _sc_scatter_add(msg, dst, H, z16)
    h1, xj = _sc_combine_gather(agg, rt, src)

    msg, rt = _dense_msgs(attr, xj, en2_W1, en2_b1, en2_W2, en2_b2, r1, s1, H,
                          h1, root2, bias2)
    agg = _sc_scatter_add(msg, dst, H, z16)
    h2, xj = _sc_combine_gather(agg, rt, src)

    msg = _dense_msgs(attr, xj, en3_W1, en3_b1, en3_W2, en3_b2, r3, s3, OUT)
    agg = _sc_scatter_add(msg, dst, OUT, z32)
    return _combine_pool(agg[:, :N], h2[:N], root3, bias3, batch3)
